# Initial kernel scaffold; baseline (speedup 1.0000x reference)
import jax
import jax.numpy as jnp
from jax.experimental import pallas as pl


def kernel(x, edge_index, edge_weight, batch, W0, b0, Wh, bh, W_att):
    raise NotImplementedError("write your pallas kernel here")


# speedup vs baseline: 17.0002x; 17.0002x over previous
import functools

import jax
import jax.numpy as jnp
from jax import lax
from jax.experimental import pallas as pl
from jax.experimental.pallas import tpu as pltpu
from jax.experimental.pallas import tpu_sc as plsc

N = 100000
B = 4
NP = N // B
E = 1600000
EP = E // B
F = 16
NTILES = 32
EPT = E // NTILES
NP_PAD = 25008

_MESH = plsc.VectorSubcoreMesh(core_axis_name="c", subcore_axis_name="s")


def _wid():
    return lax.axis_index("c") * 16 + lax.axis_index("s")


def _zero_f32(ref, nwords):
    z = jnp.zeros((16,), jnp.float32)

    def body(i, _):
        ref[pl.ds(i * 16, 16)] = z
        return 0

    lax.fori_loop(0, nwords // 16, body, 0)


_CH_A = 2000


@functools.partial(
    pl.kernel,
    out_type=jax.ShapeDtypeStruct((NTILES * NP,), jnp.float32),
    scratch_types=[
        pltpu.VMEM((NP_PAD,), jnp.float32),
        pltpu.VMEM((_CH_A,), jnp.int32),
        pltpu.VMEM((_CH_A,), jnp.float32),
    ],
    mesh=_MESH,
    compiler_params=pltpu.CompilerParams(needs_layout_passes=False),
)
def _deg_kernel(ei_hbm, ew_hbm, out_hbm, deg_loc, dbuf, wbuf):
    t = _wid()
    g = t // 8
    base = g * NP
    e0 = t * EPT
    _zero_f32(deg_loc, NP_PAD)

    def chunk(c, _):
        off = e0 + c * _CH_A
        pltpu.sync_copy(ei_hbm.at[pl.ds(E + off, _CH_A)], dbuf)
        pltpu.sync_copy(ew_hbm.at[pl.ds(off, _CH_A)], wbuf)

        def inner(i, _):
            sl = pl.ds(i * 16, 16)
            d = dbuf[sl] - base
            w = wbuf[sl]
            plsc.addupdate_scatter(deg_loc, [d], w)
            return 0

        lax.fori_loop(0, _CH_A // 16, inner, 0)
        return 0

    lax.fori_loop(0, EPT // _CH_A, chunk, 0)
    pltpu.sync_copy(deg_loc.at[pl.ds(0, NP)], out_hbm.at[pl.ds(t * NP, NP)])


def _merge_deg_body(part_ref, is_ref, sc_ref):
    deg = jnp.sum(part_ref[...], axis=1, keepdims=True) + 1.0
    inv = lax.rsqrt(deg)
    is_ref[...] = inv
    sc_ref[...] = inv * inv


def _merge_deg(deg_part):
    part = deg_part.reshape(B, 8, NP)
    return pl.pallas_call(
        _merge_deg_body,
        grid=(B,),
        in_specs=[pl.BlockSpec((1, 8, NP), lambda g: (g, 0, 0))],
        out_specs=[
            pl.BlockSpec((1, 1, NP), lambda g: (g, 0, 0)),
            pl.BlockSpec((1, 1, NP), lambda g: (g, 0, 0)),
        ],
        out_shape=[
            jax.ShapeDtypeStruct((B, 1, NP), jnp.float32),
            jax.ShapeDtypeStruct((B, 1, NP), jnp.float32),
        ],
    )(part)


_CH_C = 2000


@functools.partial(
    pl.kernel,
    out_type=(
        jax.ShapeDtypeStruct((E,), jnp.float32),
        jax.ShapeDtypeStruct((NTILES * NP,), jnp.float32),
    ),
    scratch_types=[
        pltpu.VMEM((NP_PAD,), jnp.float32),
        pltpu.VMEM((NP_PAD,), jnp.float32),
        pltpu.VMEM((NP_PAD,), jnp.float32),
        pltpu.VMEM((_CH_C,), jnp.int32),
        pltpu.VMEM((_CH_C,), jnp.int32),
        pltpu.VMEM((_CH_C,), jnp.float32),
        pltpu.VMEM((_CH_C,), jnp.float32),
    ],
    mesh=_MESH,
    compiler_params=pltpu.CompilerParams(needs_layout_passes=False),
)
def _norm_kernel(ei_hbm, ew_hbm, is_hbm, x_hbm, norm_hbm, agg0_hbm,
                 is_loc, x_loc, agg0_loc, sbuf, dbuf, wbuf, nbuf):
    t = _wid()
    g = t // 8
    base = g * NP
    e0 = t * EPT
    pltpu.sync_copy(is_hbm.at[pl.ds(base, NP)], is_loc.at[pl.ds(0, NP)])
    pltpu.sync_copy(x_hbm.at[pl.ds(base, NP)], x_loc.at[pl.ds(0, NP)])
    _zero_f32(agg0_loc, NP_PAD)

    def chunk(c, _):
        off = e0 + c * _CH_C
        pltpu.sync_copy(ei_hbm.at[pl.ds(off, _CH_C)], sbuf)
        pltpu.sync_copy(ei_hbm.at[pl.ds(E + off, _CH_C)], dbuf)
        pltpu.sync_copy(ew_hbm.at[pl.ds(off, _CH_C)], wbuf)

        def inner(i, _):
            sl = pl.ds(i * 16, 16)
            s = sbuf[sl] - base
            d = dbuf[sl] - base
            a = plsc.load_gather(is_loc, [s])
            b = plsc.load_gather(is_loc, [d])
            nv = wbuf[sl] * a * b
            nbuf[sl] = nv
            xv = plsc.load_gather(x_loc, [s])
            plsc.addupdate_scatter(agg0_loc, [d], nv * xv)
            return 0

        lax.fori_loop(0, _CH_C // 16, inner, 0)
        pltpu.sync_copy(nbuf, norm_hbm.at[pl.ds(off, _CH_C)])
        return 0

    lax.fori_loop(0, EPT // _CH_C, chunk, 0)
    pltpu.sync_copy(agg0_loc.at[pl.ds(0, NP)],
                    agg0_hbm.at[pl.ds(t * NP, NP)])


def _layer0_body(part_ref, x_ref, sc_ref, w0_ref, b0_ref, out_ref):
    agg = (jnp.sum(part_ref[...], axis=1, keepdims=True)
           + sc_ref[...] * x_ref[...])
    h = w0_ref[...] * agg + b0_ref[...]
    out_ref[...] = jnp.maximum(h, 0.0)


def _layer0(agg0_part, x3, selfc3, w0col, b0col):
    part = agg0_part.reshape(B, 8, NP)
    return pl.pallas_call(
        _layer0_body,
        grid=(B,),
        in_specs=[
            pl.BlockSpec((1, 8, NP), lambda g: (g, 0, 0)),
            pl.BlockSpec((1, 1, NP), lambda g: (g, 0, 0)),
            pl.BlockSpec((1, 1, NP), lambda g: (g, 0, 0)),
            pl.BlockSpec((F, 1), lambda g: (0, 0)),
            pl.BlockSpec((F, 1), lambda g: (0, 0)),
        ],
        out_specs=pl.BlockSpec((1, F, NP), lambda g: (g, 0, 0)),
        out_shape=jax.ShapeDtypeStruct((B, F, NP), jnp.float32),
    )(part, x3, selfc3, w0col, b0col)


_CH_E = 4000


@functools.partial(
    pl.kernel,
    out_type=jax.ShapeDtypeStruct((B * F * NP,), jnp.float32),
    scratch_types=[
        pltpu.VMEM((NP_PAD,), jnp.float32),
        pltpu.VMEM((NP_PAD,), jnp.float32),
        pltpu.VMEM((NP_PAD,), jnp.float32),
        pltpu.VMEM((NP_PAD,), jnp.float32),
        pltpu.VMEM((_CH_E,), jnp.int32),
        pltpu.VMEM((_CH_E,), jnp.int32),
        pltpu.VMEM((_CH_E,), jnp.float32),
    ],
    mesh=_MESH,
    compiler_params=pltpu.CompilerParams(needs_layout_passes=False),
)
def _agg_kernel(ht_hbm, ei_hbm, nrm_hbm, out_hbm,
                h0_loc, h1_loc, a0_loc, a1_loc, sbuf, dbuf, nbuf):
    t = _wid()
    g = t // 8
    p = t % 8
    r0 = (g * F + 2 * p) * NP
    base = g * NP
    pltpu.sync_copy(ht_hbm.at[pl.ds(r0, NP)], h0_loc.at[pl.ds(0, NP)])
    pltpu.sync_copy(ht_hbm.at[pl.ds(r0 + NP, NP)], h1_loc.at[pl.ds(0, NP)])
    _zero_f32(a0_loc, NP_PAD)
    _zero_f32(a1_loc, NP_PAD)

    def chunk(c, _):
        off = g * EP + c * _CH_E
        pltpu.sync_copy(ei_hbm.at[pl.ds(off, _CH_E)], sbuf)
        pltpu.sync_copy(ei_hbm.at[pl.ds(E + off, _CH_E)], dbuf)
        pltpu.sync_copy(nrm_hbm.at[pl.ds(off, _CH_E)], nbuf)

        def inner(i, _):
            sl = pl.ds(i * 16, 16)
            s = sbuf[sl] - base
            d = dbuf[sl] - base
            nv = nbuf[sl]
            m0 = plsc.load_gather(h0_loc, [s]) * nv
            m1 = plsc.load_gather(h1_loc, [s]) * nv
            plsc.addupdate_scatter(a0_loc, [d], m0)
            plsc.addupdate_scatter(a1_loc, [d], m1)
            return 0

        lax.fori_loop(0, _CH_E // 16, inner, 0)
        return 0

    lax.fori_loop(0, EP // _CH_E, chunk, 0)
    pltpu.sync_copy(a0_loc.at[pl.ds(0, NP)], out_hbm.at[pl.ds(r0, NP)])
    pltpu.sync_copy(a1_loc.at[pl.ds(0, NP)], out_hbm.at[pl.ds(r0 + NP, NP)])


def _mm_body(agg_ref, ht_ref, sc_ref, wt_ref, b_ref, out_ref, *, relu):
    a2 = agg_ref[...].reshape(F, NP)
    h2 = ht_ref[...].reshape(F, NP)
    sc2 = sc_ref[...].reshape(1, NP)
    tmp = a2 + sc2 * h2
    h = jnp.dot(wt_ref[...], tmp, preferred_element_type=jnp.float32)
    h = h + b_ref[...]
    if relu:
        h = jnp.maximum(h, 0.0)
    out_ref[...] = h[None]


def _matmul(aggT, hT, selfc3, wt, bcol, relu):
    return pl.pallas_call(
        functools.partial(_mm_body, relu=relu),
        grid=(B,),
        in_specs=[
            pl.BlockSpec((1, F, NP), lambda g: (g, 0, 0)),
            pl.BlockSpec((1, F, NP), lambda g: (g, 0, 0)),
            pl.BlockSpec((1, 1, NP), lambda g: (g, 0, 0)),
            pl.BlockSpec((F, F), lambda g: (0, 0)),
            pl.BlockSpec((F, 1), lambda g: (0, 0)),
        ],
        out_specs=pl.BlockSpec((1, F, NP), lambda g: (g, 0, 0)),
        out_shape=jax.ShapeDtypeStruct((B, F, NP), jnp.float32),
    )(aggT, hT, selfc3, wt, bcol)


def _gsum_body(ht_ref, out_ref):
    out_ref[...] = jnp.sum(ht_ref[...], axis=2, keepdims=True)


def _gsum(hT3):
    return pl.pallas_call(
        _gsum_body,
        grid=(B,),
        in_specs=[pl.BlockSpec((1, F, NP), lambda g: (g, 0, 0))],
        out_specs=pl.BlockSpec((1, F, 1), lambda g: (g, 0, 0)),
        out_shape=jax.ShapeDtypeStruct((B, F, 1), jnp.float32),
    )(hT3)


def _pool_body(ht_ref, gs_ref, watt_ref, out_ref):
    g = pl.program_id(0)
    gmean = gs_ref[...].reshape(B, F) * (1.0 / NP)
    ctx = jnp.tanh(jnp.dot(gmean, watt_ref[...],
                           preferred_element_type=jnp.float32))
    sel = (lax.broadcasted_iota(jnp.int32, (B, 1), 0) == g).astype(jnp.float32)
    cg = jnp.sum(ctx * sel, axis=0, keepdims=True).reshape(F, 1)
    h = ht_ref[...].reshape(F, NP)
    scores = jax.nn.sigmoid(jnp.sum(h * cg, axis=0, keepdims=True))
    gf = jnp.sum(h * scores, axis=1, keepdims=True)
    node = h.T
    gfeat = jnp.broadcast_to(gf.reshape(1, F), (NP, F))
    out_ref[...] = jnp.concatenate([node, gfeat], axis=1)[None]


def _pool(hT3, gsum, watt):
    return pl.pallas_call(
        _pool_body,
        grid=(B,),
        in_specs=[
            pl.BlockSpec((1, F, NP), lambda g: (g, 0, 0)),
            pl.BlockSpec((B, F, 1), lambda g: (0, 0, 0)),
            pl.BlockSpec((F, F), lambda g: (0, 0)),
        ],
        out_specs=pl.BlockSpec((1, NP, 2 * F), lambda g: (g, 0, 0)),
        out_shape=jax.ShapeDtypeStruct((B, NP, 2 * F), jnp.float32),
    )(hT3, gsum, watt)


def kernel(x, edge_index, edge_weight, batch, W0, b0, Wh, bh, W_att):
    assert x.shape == (N, 1) and edge_index.shape == (2, E)
    LH = Wh.shape[0]

    ei_flat = edge_index.reshape(2 * E)
    x_flat = x.reshape(N)
    x3 = x.reshape(B, 1, NP)

    deg_part = _deg_kernel(ei_flat, edge_weight)
    is3, selfc3 = _merge_deg(deg_part)
    is_flat = is3.reshape(N)

    norm, agg0_part = _norm_kernel(ei_flat, edge_weight, is_flat, x_flat)

    w0col = W0.reshape(F, 1)
    b0col = b0.reshape(F, 1)
    hT3 = _layer0(agg0_part, x3, selfc3, w0col, b0col)

    for i in range(LH):
        aggT_flat = _agg_kernel(hT3.reshape(B * F * NP), ei_flat, norm)
        aggT3 = aggT_flat.reshape(B, F, NP)
        wt = Wh[i].T
        bcol = bh[i].reshape(F, 1)
        hT3 = _matmul(aggT3, hT3, selfc3, wt, bcol, relu=(i < LH - 1))

    gsum = _gsum(hT3)
    state = _pool(hT3, gsum, W_att)
    return state


# speedup vs baseline: 47.0159x; 2.7656x over previous
import functools

import jax
import jax.numpy as jnp
from jax import lax
from jax.experimental import pallas as pl
from jax.experimental.pallas import tpu as pltpu
from jax.experimental.pallas import tpu_sc as plsc

N = 100000
B = 4
NP = N // B
E = 1600000
EP = E // B
F = 16
NTILES = 32
EPT = E // NTILES
NP_PAD = 25008

_MESH = plsc.VectorSubcoreMesh(core_axis_name="c", subcore_axis_name="s")


def _wid():
    return lax.axis_index("c") * 16 + lax.axis_index("s")


def _zero_f32(ref, nwords):
    z = jnp.zeros((16,), jnp.float32)

    @plsc.parallel_loop(0, nwords // 16, unroll=8)
    def body(i):
        ref[pl.ds(i * 16, 16)] = z


_CH_A = 2000


@functools.partial(
    pl.kernel,
    out_type=jax.ShapeDtypeStruct((NTILES * NP,), jnp.float32),
    scratch_types=[
        pltpu.VMEM((NP_PAD,), jnp.float32),
        pltpu.VMEM((_CH_A,), jnp.int32),
        pltpu.VMEM((_CH_A,), jnp.float32),
    ],
    mesh=_MESH,
    compiler_params=pltpu.CompilerParams(needs_layout_passes=False),
)
def _deg_kernel(ei_hbm, ew_hbm, out_hbm, deg_loc, dbuf, wbuf):
    t = _wid()
    g = t // 8
    base = g * NP
    e0 = t * EPT
    _zero_f32(deg_loc, NP_PAD)

    def chunk(c, _):
        off = e0 + c * _CH_A
        pltpu.sync_copy(ei_hbm.at[pl.ds(E + off, _CH_A)], dbuf)
        pltpu.sync_copy(ew_hbm.at[pl.ds(off, _CH_A)], wbuf)

        @plsc.parallel_loop(0, _CH_A // 16, unroll=8)
        def inner(i):
            sl = pl.ds(i * 16, 16)
            d = dbuf[sl] - base
            w = wbuf[sl]
            plsc.addupdate_scatter(deg_loc, [d], w)

        return 0

    lax.fori_loop(0, EPT // _CH_A, chunk, 0)
    pltpu.sync_copy(deg_loc.at[pl.ds(0, NP)], out_hbm.at[pl.ds(t * NP, NP)])


def _merge_deg_body(part_ref, is_ref, sc_ref):
    deg = jnp.sum(part_ref[...], axis=1, keepdims=True) + 1.0
    inv = lax.rsqrt(deg)
    is_ref[...] = inv
    sc_ref[...] = inv * inv


def _merge_deg(deg_part):
    part = deg_part.reshape(B, 8, NP)
    return pl.pallas_call(
        _merge_deg_body,
        grid=(B,),
        in_specs=[pl.BlockSpec((1, 8, NP), lambda g: (g, 0, 0))],
        out_specs=[
            pl.BlockSpec((1, 1, NP), lambda g: (g, 0, 0)),
            pl.BlockSpec((1, 1, NP), lambda g: (g, 0, 0)),
        ],
        out_shape=[
            jax.ShapeDtypeStruct((B, 1, NP), jnp.float32),
            jax.ShapeDtypeStruct((B, 1, NP), jnp.float32),
        ],
    )(part)


_CH_C = 2000


@functools.partial(
    pl.kernel,
    out_type=(
        jax.ShapeDtypeStruct((E,), jnp.float32),
        jax.ShapeDtypeStruct((NTILES * NP,), jnp.float32),
    ),
    scratch_types=[
        pltpu.VMEM((NP_PAD,), jnp.float32),
        pltpu.VMEM((NP_PAD,), jnp.float32),
        pltpu.VMEM((NP_PAD,), jnp.float32),
        pltpu.VMEM((_CH_C,), jnp.int32),
        pltpu.VMEM((_CH_C,), jnp.int32),
        pltpu.VMEM((_CH_C,), jnp.float32),
        pltpu.VMEM((_CH_C,), jnp.float32),
    ],
    mesh=_MESH,
    compiler_params=pltpu.CompilerParams(needs_layout_passes=False),
)
def _norm_kernel(ei_hbm, ew_hbm, is_hbm, x_hbm, norm_hbm, agg0_hbm,
                 is_loc, x_loc, agg0_loc, sbuf, dbuf, wbuf, nbuf):
    t = _wid()
    g = t // 8
    base = g * NP
    e0 = t * EPT
    pltpu.sync_copy(is_hbm.at[pl.ds(base, NP)], is_loc.at[pl.ds(0, NP)])
    pltpu.sync_copy(x_hbm.at[pl.ds(base, NP)], x_loc.at[pl.ds(0, NP)])
    _zero_f32(agg0_loc, NP_PAD)

    def chunk(c, _):
        off = e0 + c * _CH_C
        pltpu.sync_copy(ei_hbm.at[pl.ds(off, _CH_C)], sbuf)
        pltpu.sync_copy(ei_hbm.at[pl.ds(E + off, _CH_C)], dbuf)
        pltpu.sync_copy(ew_hbm.at[pl.ds(off, _CH_C)], wbuf)

        @plsc.parallel_loop(0, _CH_C // 16, unroll=8)
        def inner(i):
            sl = pl.ds(i * 16, 16)
            s = sbuf[sl] - base
            d = dbuf[sl] - base
            a = plsc.load_gather(is_loc, [s])
            b = plsc.load_gather(is_loc, [d])
            nv = wbuf[sl] * a * b
            nbuf[sl] = nv
            xv = plsc.load_gather(x_loc, [s])
            plsc.addupdate_scatter(agg0_loc, [d], nv * xv)
        pltpu.sync_copy(nbuf, norm_hbm.at[pl.ds(off, _CH_C)])
        return 0

    lax.fori_loop(0, EPT // _CH_C, chunk, 0)
    pltpu.sync_copy(agg0_loc.at[pl.ds(0, NP)],
                    agg0_hbm.at[pl.ds(t * NP, NP)])


def _layer0_body(part_ref, x_ref, sc_ref, w0_ref, b0_ref, out_ref):
    agg = (jnp.sum(part_ref[...], axis=1, keepdims=True)
           + sc_ref[...] * x_ref[...])
    h = w0_ref[...] * agg + b0_ref[...]
    out_ref[...] = jnp.maximum(h, 0.0)


def _layer0(agg0_part, x3, selfc3, w0col, b0col):
    part = agg0_part.reshape(B, 8, NP)
    return pl.pallas_call(
        _layer0_body,
        grid=(B,),
        in_specs=[
            pl.BlockSpec((1, 8, NP), lambda g: (g, 0, 0)),
            pl.BlockSpec((1, 1, NP), lambda g: (g, 0, 0)),
            pl.BlockSpec((1, 1, NP), lambda g: (g, 0, 0)),
            pl.BlockSpec((F, 1), lambda g: (0, 0)),
            pl.BlockSpec((F, 1), lambda g: (0, 0)),
        ],
        out_specs=pl.BlockSpec((1, F, NP), lambda g: (g, 0, 0)),
        out_shape=jax.ShapeDtypeStruct((B, F, NP), jnp.float32),
    )(part, x3, selfc3, w0col, b0col)


_CH_E = 2000
_NCH_E = EP // _CH_E
_NPAIR = _NCH_E // 2


@functools.partial(
    pl.kernel,
    out_type=jax.ShapeDtypeStruct((B * F * NP,), jnp.float32),
    scratch_types=[
        pltpu.VMEM((NP_PAD,), jnp.float32),
        pltpu.VMEM((NP_PAD,), jnp.float32),
        pltpu.VMEM((NP_PAD,), jnp.float32),
        pltpu.VMEM((NP_PAD,), jnp.float32),
        pltpu.VMEM((_CH_E,), jnp.int32),
        pltpu.VMEM((_CH_E,), jnp.int32),
        pltpu.VMEM((_CH_E,), jnp.float32),
        pltpu.VMEM((_CH_E,), jnp.int32),
        pltpu.VMEM((_CH_E,), jnp.int32),
        pltpu.VMEM((_CH_E,), jnp.float32),
        pltpu.SemaphoreType.DMA,
        pltpu.SemaphoreType.DMA,
    ],
    mesh=_MESH,
    compiler_params=pltpu.CompilerParams(needs_layout_passes=False),
)
def _agg_kernel(ht_hbm, ei_hbm, nrm_hbm, out_hbm,
                h0_loc, h1_loc, a0_loc, a1_loc,
                sb0, db0, nb0, sb1, db1, nb1, sem0, sem1):
    t = _wid()
    g = t // 8
    p = t % 8
    r0 = (g * F + 2 * p) * NP
    base = g * NP
    pltpu.sync_copy(ht_hbm.at[pl.ds(r0, NP)], h0_loc.at[pl.ds(0, NP)])
    pltpu.sync_copy(ht_hbm.at[pl.ds(r0 + NP, NP)], h1_loc.at[pl.ds(0, NP)])
    _zero_f32(a0_loc, NP_PAD)
    _zero_f32(a1_loc, NP_PAD)

    def start(c, sb, db, nb, sem):
        off = g * EP + c * _CH_E
        pltpu.make_async_copy(ei_hbm.at[pl.ds(off, _CH_E)], sb, sem).start()
        pltpu.make_async_copy(
            ei_hbm.at[pl.ds(E + off, _CH_E)], db, sem).start()
        pltpu.make_async_copy(nrm_hbm.at[pl.ds(off, _CH_E)], nb, sem).start()

    def wait(sb, db, nb, sem):
        pltpu.make_async_copy(ei_hbm.at[pl.ds(0, _CH_E)], sb, sem).wait()
        pltpu.make_async_copy(ei_hbm.at[pl.ds(0, _CH_E)], db, sem).wait()
        pltpu.make_async_copy(nrm_hbm.at[pl.ds(0, _CH_E)], nb, sem).wait()

    def edge_pass(sb, db, nb):
        @plsc.parallel_loop(0, _CH_E // 16, unroll=8)
        def inner(i):
            sl = pl.ds(i * 16, 16)
            s = sb[sl] - base
            d = db[sl] - base
            nv = nb[sl]
            m0 = plsc.load_gather(h0_loc, [s]) * nv
            m1 = plsc.load_gather(h1_loc, [s]) * nv
            plsc.addupdate_scatter(a0_loc, [d], m0)
            plsc.addupdate_scatter(a1_loc, [d], m1)

    start(0, sb0, db0, nb0, sem0)

    def pair(c2, _):
        c = 2 * c2
        start(c + 1, sb1, db1, nb1, sem1)
        wait(sb0, db0, nb0, sem0)
        edge_pass(sb0, db0, nb0)

        @pl.when(c2 + 1 < _NPAIR)
        def _():
            start(c + 2, sb0, db0, nb0, sem0)

        wait(sb1, db1, nb1, sem1)
        edge_pass(sb1, db1, nb1)
        return 0

    lax.fori_loop(0, _NPAIR, pair, 0)
    pltpu.sync_copy(a0_loc.at[pl.ds(0, NP)], out_hbm.at[pl.ds(r0, NP)])
    pltpu.sync_copy(a1_loc.at[pl.ds(0, NP)], out_hbm.at[pl.ds(r0 + NP, NP)])


def _mm_body(agg_ref, ht_ref, sc_ref, wt_ref, b_ref, out_ref, *, relu):
    a2 = agg_ref[...].reshape(F, NP)
    h2 = ht_ref[...].reshape(F, NP)
    sc2 = sc_ref[...].reshape(1, NP)
    tmp = a2 + sc2 * h2
    h = jnp.dot(wt_ref[...], tmp, preferred_element_type=jnp.float32)
    h = h + b_ref[...]
    if relu:
        h = jnp.maximum(h, 0.0)
    out_ref[...] = h[None]


def _matmul(aggT, hT, selfc3, wt, bcol, relu):
    return pl.pallas_call(
        functools.partial(_mm_body, relu=relu),
        grid=(B,),
        in_specs=[
            pl.BlockSpec((1, F, NP), lambda g: (g, 0, 0)),
            pl.BlockSpec((1, F, NP), lambda g: (g, 0, 0)),
            pl.BlockSpec((1, 1, NP), lambda g: (g, 0, 0)),
            pl.BlockSpec((F, F), lambda g: (0, 0)),
            pl.BlockSpec((F, 1), lambda g: (0, 0)),
        ],
        out_specs=pl.BlockSpec((1, F, NP), lambda g: (g, 0, 0)),
        out_shape=jax.ShapeDtypeStruct((B, F, NP), jnp.float32),
    )(aggT, hT, selfc3, wt, bcol)


def _gsum_body(ht_ref, out_ref):
    out_ref[...] = jnp.sum(ht_ref[...], axis=2, keepdims=True)


def _gsum(hT3):
    return pl.pallas_call(
        _gsum_body,
        grid=(B,),
        in_specs=[pl.BlockSpec((1, F, NP), lambda g: (g, 0, 0))],
        out_specs=pl.BlockSpec((1, F, 1), lambda g: (g, 0, 0)),
        out_shape=jax.ShapeDtypeStruct((B, F, 1), jnp.float32),
    )(hT3)


def _pool_body(ht_ref, gs_ref, watt_ref, out_ref):
    g = pl.program_id(0)
    gmean = gs_ref[...].reshape(B, F) * (1.0 / NP)
    ctx = jnp.tanh(jnp.dot(gmean, watt_ref[...],
                           preferred_element_type=jnp.float32))
    sel = (lax.broadcasted_iota(jnp.int32, (B, 1), 0) == g).astype(jnp.float32)
    cg = jnp.sum(ctx * sel, axis=0, keepdims=True).reshape(F, 1)
    h = ht_ref[...].reshape(F, NP)
    scores = jax.nn.sigmoid(jnp.sum(h * cg, axis=0, keepdims=True))
    gf = jnp.sum(h * scores, axis=1, keepdims=True)
    node = h.T
    gfeat = jnp.broadcast_to(gf.reshape(1, F), (NP, F))
    out_ref[...] = jnp.concatenate([node, gfeat], axis=1)[None]


def _pool(hT3, gsum, watt):
    return pl.pallas_call(
        _pool_body,
        grid=(B,),
        in_specs=[
            pl.BlockSpec((1, F, NP), lambda g: (g, 0, 0)),
            pl.BlockSpec((B, F, 1), lambda g: (0, 0, 0)),
            pl.BlockSpec((F, F), lambda g: (0, 0)),
        ],
        out_specs=pl.BlockSpec((1, NP, 2 * F), lambda g: (g, 0, 0)),
        out_shape=jax.ShapeDtypeStruct((B, NP, 2 * F), jnp.float32),
    )(hT3, gsum, watt)


def kernel(x, edge_index, edge_weight, batch, W0, b0, Wh, bh, W_att):
    assert x.shape == (N, 1) and edge_index.shape == (2, E)
    LH = Wh.shape[0]

    ei_flat = edge_index.reshape(2 * E)
    x_flat = x.reshape(N)
    x3 = x.reshape(B, 1, NP)

    deg_part = _deg_kernel(ei_flat, edge_weight)
    is3, selfc3 = _merge_deg(deg_part)
    is_flat = is3.reshape(N)

    norm, agg0_part = _norm_kernel(ei_flat, edge_weight, is_flat, x_flat)

    w0col = W0.reshape(F, 1)
    b0col = b0.reshape(F, 1)
    hT3 = _layer0(agg0_part, x3, selfc3, w0col, b0col)

    for i in range(LH):
        aggT_flat = _agg_kernel(hT3.reshape(B * F * NP), ei_flat, norm)
        aggT3 = aggT_flat.reshape(B, F, NP)
        wt = Wh[i].T
        bcol = bh[i].reshape(F, 1)
        hT3 = _matmul(aggT3, hT3, selfc3, wt, bcol, relu=(i < LH - 1))

    gsum = _gsum(hT3)
    state = _pool(hT3, gsum, W_att)
    return state


# speedup vs baseline: 50.4248x; 1.0725x over previous
import functools

import jax
import jax.numpy as jnp
from jax import lax
from jax.experimental import pallas as pl
from jax.experimental.pallas import tpu as pltpu
from jax.experimental.pallas import tpu_sc as plsc

N = 100000
B = 4
NP = N // B
E = 1600000
EP = E // B
F = 16
NTILES = 32
EPT = E // NTILES
NP_PAD = 25008

_MESH = plsc.VectorSubcoreMesh(core_axis_name="c", subcore_axis_name="s")


def _wid():
    return lax.axis_index("c") * 16 + lax.axis_index("s")


def _zero_f32(ref, nwords):
    z = jnp.zeros((16,), jnp.float32)

    @plsc.parallel_loop(0, nwords // 16, unroll=8)
    def body(i):
        ref[pl.ds(i * 16, 16)] = z


_CH_A = 2000


@functools.partial(
    pl.kernel,
    out_type=jax.ShapeDtypeStruct((NTILES * NP,), jnp.float32),
    scratch_types=[
        pltpu.VMEM((NP_PAD,), jnp.float32),
        pltpu.VMEM((_CH_A,), jnp.int32),
        pltpu.VMEM((_CH_A,), jnp.float32),
    ],
    mesh=_MESH,
    compiler_params=pltpu.CompilerParams(needs_layout_passes=False),
)
def _deg_kernel(ei_hbm, ew_hbm, out_hbm, deg_loc, dbuf, wbuf):
    t = _wid()
    g = t // 8
    base = g * NP
    e0 = t * EPT
    _zero_f32(deg_loc, NP_PAD)

    def chunk(c, _):
        off = e0 + c * _CH_A
        pltpu.sync_copy(ei_hbm.at[pl.ds(E + off, _CH_A)], dbuf)
        pltpu.sync_copy(ew_hbm.at[pl.ds(off, _CH_A)], wbuf)

        @plsc.parallel_loop(0, _CH_A // 16, unroll=8)
        def inner(i):
            sl = pl.ds(i * 16, 16)
            d = dbuf[sl] - base
            w = wbuf[sl]
            plsc.addupdate_scatter(deg_loc, [d], w)

        return 0

    lax.fori_loop(0, EPT // _CH_A, chunk, 0)
    pltpu.sync_copy(deg_loc.at[pl.ds(0, NP)], out_hbm.at[pl.ds(t * NP, NP)])


def _merge_deg_body(part_ref, is_ref, sc_ref):
    deg = jnp.sum(part_ref[...], axis=1, keepdims=True) + 1.0
    inv = lax.rsqrt(deg)
    is_ref[...] = inv
    sc_ref[...] = inv * inv


def _merge_deg(deg_part):
    part = deg_part.reshape(B, 8, NP)
    return pl.pallas_call(
        _merge_deg_body,
        grid=(B,),
        in_specs=[pl.BlockSpec((1, 8, NP), lambda g: (g, 0, 0))],
        out_specs=[
            pl.BlockSpec((1, 1, NP), lambda g: (g, 0, 0)),
            pl.BlockSpec((1, 1, NP), lambda g: (g, 0, 0)),
        ],
        out_shape=[
            jax.ShapeDtypeStruct((B, 1, NP), jnp.float32),
            jax.ShapeDtypeStruct((B, 1, NP), jnp.float32),
        ],
    )(part)


_CH_C = 2000


@functools.partial(
    pl.kernel,
    out_type=(
        jax.ShapeDtypeStruct((E,), jnp.float32),
        jax.ShapeDtypeStruct((E,), jnp.int32),
        jax.ShapeDtypeStruct((NTILES * NP,), jnp.float32),
    ),
    scratch_types=[
        pltpu.VMEM((NP_PAD,), jnp.float32),
        pltpu.VMEM((NP_PAD,), jnp.float32),
        pltpu.VMEM((NP_PAD,), jnp.float32),
        pltpu.VMEM((_CH_C,), jnp.int32),
        pltpu.VMEM((_CH_C,), jnp.int32),
        pltpu.VMEM((_CH_C,), jnp.float32),
        pltpu.VMEM((_CH_C,), jnp.float32),
        pltpu.VMEM((_CH_C,), jnp.int32),
    ],
    mesh=_MESH,
    compiler_params=pltpu.CompilerParams(needs_layout_passes=False),
)
def _norm_kernel(ei_hbm, ew_hbm, is_hbm, x_hbm, norm_hbm, pk_hbm, agg0_hbm,
                 is_loc, x_loc, agg0_loc, sbuf, dbuf, wbuf, nbuf, pkbuf):
    t = _wid()
    g = t // 8
    base = g * NP
    e0 = t * EPT
    pltpu.sync_copy(is_hbm.at[pl.ds(base, NP)], is_loc.at[pl.ds(0, NP)])
    pltpu.sync_copy(x_hbm.at[pl.ds(base, NP)], x_loc.at[pl.ds(0, NP)])
    _zero_f32(agg0_loc, NP_PAD)

    def chunk(c, _):
        off = e0 + c * _CH_C
        pltpu.sync_copy(ei_hbm.at[pl.ds(off, _CH_C)], sbuf)
        pltpu.sync_copy(ei_hbm.at[pl.ds(E + off, _CH_C)], dbuf)
        pltpu.sync_copy(ew_hbm.at[pl.ds(off, _CH_C)], wbuf)

        @plsc.parallel_loop(0, _CH_C // 16, unroll=8)
        def inner(i):
            sl = pl.ds(i * 16, 16)
            s = sbuf[sl] - base
            d = dbuf[sl] - base
            pkbuf[sl] = s | (d << 16)
            a = plsc.load_gather(is_loc, [s])
            b = plsc.load_gather(is_loc, [d])
            nv = wbuf[sl] * a * b
            nbuf[sl] = nv
            xv = plsc.load_gather(x_loc, [s])
            plsc.addupdate_scatter(agg0_loc, [d], nv * xv)
        pltpu.sync_copy(nbuf, norm_hbm.at[pl.ds(off, _CH_C)])
        pltpu.sync_copy(pkbuf, pk_hbm.at[pl.ds(off, _CH_C)])
        return 0

    lax.fori_loop(0, EPT // _CH_C, chunk, 0)
    pltpu.sync_copy(agg0_loc.at[pl.ds(0, NP)],
                    agg0_hbm.at[pl.ds(t * NP, NP)])


def _layer0_body(part_ref, x_ref, sc_ref, w0_ref, b0_ref, out_ref):
    agg = (jnp.sum(part_ref[...], axis=1, keepdims=True)
           + sc_ref[...] * x_ref[...])
    h = w0_ref[...] * agg + b0_ref[...]
    out_ref[...] = jnp.maximum(h, 0.0)


def _layer0(agg0_part, x3, selfc3, w0col, b0col):
    part = agg0_part.reshape(B, 8, NP)
    return pl.pallas_call(
        _layer0_body,
        grid=(B,),
        in_specs=[
            pl.BlockSpec((1, 8, NP), lambda g: (g, 0, 0)),
            pl.BlockSpec((1, 1, NP), lambda g: (g, 0, 0)),
            pl.BlockSpec((1, 1, NP), lambda g: (g, 0, 0)),
            pl.BlockSpec((F, 1), lambda g: (0, 0)),
            pl.BlockSpec((F, 1), lambda g: (0, 0)),
        ],
        out_specs=pl.BlockSpec((1, F, NP), lambda g: (g, 0, 0)),
        out_shape=jax.ShapeDtypeStruct((B, F, NP), jnp.float32),
    )(part, x3, selfc3, w0col, b0col)


_CH_E = 2000
_NCH_E = EP // _CH_E
_NPAIR = _NCH_E // 2


@functools.partial(
    pl.kernel,
    out_type=jax.ShapeDtypeStruct((B * F * NP,), jnp.float32),
    scratch_types=[
        pltpu.VMEM((NP_PAD,), jnp.float32),
        pltpu.VMEM((NP_PAD,), jnp.float32),
        pltpu.VMEM((NP_PAD,), jnp.float32),
        pltpu.VMEM((NP_PAD,), jnp.float32),
        pltpu.VMEM((_CH_E,), jnp.int32),
        pltpu.VMEM((_CH_E,), jnp.float32),
        pltpu.VMEM((_CH_E,), jnp.int32),
        pltpu.VMEM((_CH_E,), jnp.float32),
        pltpu.SemaphoreType.DMA,
        pltpu.SemaphoreType.DMA,
    ],
    mesh=_MESH,
    compiler_params=pltpu.CompilerParams(needs_layout_passes=False),
)
def _agg_kernel(ht_hbm, pk_hbm, nrm_hbm, out_hbm,
                h0_loc, h1_loc, a0_loc, a1_loc,
                pb0, nb0, pb1, nb1, sem0, sem1):
    t = _wid()
    g = t // 8
    p = t % 8
    r0 = (g * F + 2 * p) * NP
    pltpu.sync_copy(ht_hbm.at[pl.ds(r0, NP)], h0_loc.at[pl.ds(0, NP)])
    pltpu.sync_copy(ht_hbm.at[pl.ds(r0 + NP, NP)], h1_loc.at[pl.ds(0, NP)])
    _zero_f32(a0_loc, NP_PAD)
    _zero_f32(a1_loc, NP_PAD)

    def start(c, pb, nb, sem):
        off = g * EP + c * _CH_E
        pltpu.make_async_copy(pk_hbm.at[pl.ds(off, _CH_E)], pb, sem).start()
        pltpu.make_async_copy(nrm_hbm.at[pl.ds(off, _CH_E)], nb, sem).start()

    def wait(pb, nb, sem):
        pltpu.make_async_copy(pk_hbm.at[pl.ds(0, _CH_E)], pb, sem).wait()
        pltpu.make_async_copy(nrm_hbm.at[pl.ds(0, _CH_E)], nb, sem).wait()

    def edge_pass(pb, nb):
        @plsc.parallel_loop(0, _CH_E // 16, unroll=8)
        def inner(i):
            sl = pl.ds(i * 16, 16)
            pk = pb[sl]
            s = pk & 0xFFFF
            d = pk >> 16
            nv = nb[sl]
            m0 = plsc.load_gather(h0_loc, [s]) * nv
            m1 = plsc.load_gather(h1_loc, [s]) * nv
            plsc.addupdate_scatter(a0_loc, [d], m0)
            plsc.addupdate_scatter(a1_loc, [d], m1)

    start(0, pb0, nb0, sem0)

    def pair(c2, _):
        c = 2 * c2
        start(c + 1, pb1, nb1, sem1)
        wait(pb0, nb0, sem0)
        edge_pass(pb0, nb0)

        @pl.when(c2 + 1 < _NPAIR)
        def _():
            start(c + 2, pb0, nb0, sem0)

        wait(pb1, nb1, sem1)
        edge_pass(pb1, nb1)
        return 0

    lax.fori_loop(0, _NPAIR, pair, 0)
    pltpu.sync_copy(a0_loc.at[pl.ds(0, NP)], out_hbm.at[pl.ds(r0, NP)])
    pltpu.sync_copy(a1_loc.at[pl.ds(0, NP)], out_hbm.at[pl.ds(r0 + NP, NP)])


def _mm_body(agg_ref, ht_ref, sc_ref, wt_ref, b_ref, out_ref, *, relu):
    a2 = agg_ref[...].reshape(F, NP)
    h2 = ht_ref[...].reshape(F, NP)
    sc2 = sc_ref[...].reshape(1, NP)
    tmp = a2 + sc2 * h2
    h = jnp.dot(wt_ref[...], tmp, preferred_element_type=jnp.float32)
    h = h + b_ref[...]
    if relu:
        h = jnp.maximum(h, 0.0)
    out_ref[...] = h[None]


def _matmul(aggT, hT, selfc3, wt, bcol, relu):
    return pl.pallas_call(
        functools.partial(_mm_body, relu=relu),
        grid=(B,),
        in_specs=[
            pl.BlockSpec((1, F, NP), lambda g: (g, 0, 0)),
            pl.BlockSpec((1, F, NP), lambda g: (g, 0, 0)),
            pl.BlockSpec((1, 1, NP), lambda g: (g, 0, 0)),
            pl.BlockSpec((F, F), lambda g: (0, 0)),
            pl.BlockSpec((F, 1), lambda g: (0, 0)),
        ],
        out_specs=pl.BlockSpec((1, F, NP), lambda g: (g, 0, 0)),
        out_shape=jax.ShapeDtypeStruct((B, F, NP), jnp.float32),
    )(aggT, hT, selfc3, wt, bcol)


def _gsum_body(ht_ref, out_ref):
    out_ref[...] = jnp.sum(ht_ref[...], axis=2, keepdims=True)


def _gsum(hT3):
    return pl.pallas_call(
        _gsum_body,
        grid=(B,),
        in_specs=[pl.BlockSpec((1, F, NP), lambda g: (g, 0, 0))],
        out_specs=pl.BlockSpec((1, F, 1), lambda g: (g, 0, 0)),
        out_shape=jax.ShapeDtypeStruct((B, F, 1), jnp.float32),
    )(hT3)


def _pool_body(ht_ref, gs_ref, watt_ref, out_ref):
    g = pl.program_id(0)
    gmean = gs_ref[...].reshape(B, F) * (1.0 / NP)
    ctx = jnp.tanh(jnp.dot(gmean, watt_ref[...],
                           preferred_element_type=jnp.float32))
    sel = (lax.broadcasted_iota(jnp.int32, (B, 1), 0) == g).astype(jnp.float32)
    cg = jnp.sum(ctx * sel, axis=0, keepdims=True).reshape(F, 1)
    h = ht_ref[...].reshape(F, NP)
    scores = jax.nn.sigmoid(jnp.sum(h * cg, axis=0, keepdims=True))
    gf = jnp.sum(h * scores, axis=1, keepdims=True)
    node = h.T
    gfeat = jnp.broadcast_to(gf.reshape(1, F), (NP, F))
    out_ref[...] = jnp.concatenate([node, gfeat], axis=1)[None]


def _pool(hT3, gsum, watt):
    return pl.pallas_call(
        _pool_body,
        grid=(B,),
        in_specs=[
            pl.BlockSpec((1, F, NP), lambda g: (g, 0, 0)),
            pl.BlockSpec((B, F, 1), lambda g: (0, 0, 0)),
            pl.BlockSpec((F, F), lambda g: (0, 0)),
        ],
        out_specs=pl.BlockSpec((1, NP, 2 * F), lambda g: (g, 0, 0)),
        out_shape=jax.ShapeDtypeStruct((B, NP, 2 * F), jnp.float32),
    )(hT3, gsum, watt)


def kernel(x, edge_index, edge_weight, batch, W0, b0, Wh, bh, W_att):
    assert x.shape == (N, 1) and edge_index.shape == (2, E)
    LH = Wh.shape[0]

    ei_flat = edge_index.reshape(2 * E)
    x_flat = x.reshape(N)
    x3 = x.reshape(B, 1, NP)

    deg_part = _deg_kernel(ei_flat, edge_weight)
    is3, selfc3 = _merge_deg(deg_part)
    is_flat = is3.reshape(N)

    norm, pk, agg0_part = _norm_kernel(ei_flat, edge_weight, is_flat, x_flat)

    w0col = W0.reshape(F, 1)
    b0col = b0.reshape(F, 1)
    hT3 = _layer0(agg0_part, x3, selfc3, w0col, b0col)

    for i in range(LH):
        aggT_flat = _agg_kernel(hT3.reshape(B * F * NP), pk, norm)
        aggT3 = aggT_flat.reshape(B, F, NP)
        wt = Wh[i].T
        bcol = bh[i].reshape(F, 1)
        hT3 = _matmul(aggT3, hT3, selfc3, wt, bcol, relu=(i < LH - 1))

    gsum = _gsum(hT3)
    state = _pool(hT3, gsum, W_att)
    return state


# speedup vs baseline: 51.7490x; 1.0263x over previous
import functools

import jax
import jax.numpy as jnp
from jax import lax
from jax.experimental import pallas as pl
from jax.experimental.pallas import tpu as pltpu
from jax.experimental.pallas import tpu_sc as plsc

N = 100000
B = 4
NP = N // B
E = 1600000
EP = E // B
F = 16
NTILES = 32
EPT = E // NTILES
NP_PAD = 25008

_MESH = plsc.VectorSubcoreMesh(core_axis_name="c", subcore_axis_name="s")


def _wid():
    return lax.axis_index("c") * 16 + lax.axis_index("s")


def _zero_f32(ref, nwords):
    z = jnp.zeros((16,), jnp.float32)

    @plsc.parallel_loop(0, nwords // 16, unroll=8)
    def body(i):
        ref[pl.ds(i * 16, 16)] = z


_CH_A = 2000


@functools.partial(
    pl.kernel,
    out_type=jax.ShapeDtypeStruct((NTILES * NP,), jnp.float32),
    scratch_types=[
        pltpu.VMEM((NP_PAD,), jnp.float32),
        pltpu.VMEM((_CH_A,), jnp.int32),
        pltpu.VMEM((_CH_A,), jnp.float32),
    ],
    mesh=_MESH,
    compiler_params=pltpu.CompilerParams(needs_layout_passes=False),
)
def _deg_kernel(ei_hbm, ew_hbm, out_hbm, deg_loc, dbuf, wbuf):
    t = _wid()
    g = t // 8
    base = g * NP
    e0 = t * EPT
    _zero_f32(deg_loc, NP_PAD)

    def chunk(c, _):
        off = e0 + c * _CH_A
        pltpu.sync_copy(ei_hbm.at[pl.ds(E + off, _CH_A)], dbuf)
        pltpu.sync_copy(ew_hbm.at[pl.ds(off, _CH_A)], wbuf)

        @plsc.parallel_loop(0, _CH_A // 16, unroll=8)
        def inner(i):
            sl = pl.ds(i * 16, 16)
            d = dbuf[sl] - base
            w = wbuf[sl]
            plsc.addupdate_scatter(deg_loc, [d], w)

        return 0

    lax.fori_loop(0, EPT // _CH_A, chunk, 0)
    pltpu.sync_copy(deg_loc.at[pl.ds(0, NP)], out_hbm.at[pl.ds(t * NP, NP)])


def _merge_deg_body(part_ref, is_ref, sc_ref):
    deg = jnp.sum(part_ref[...], axis=1, keepdims=True) + 1.0
    inv = lax.rsqrt(deg)
    is_ref[...] = inv
    sc_ref[...] = inv * inv


def _merge_deg(deg_part):
    part = deg_part.reshape(B, 8, NP)
    return pl.pallas_call(
        _merge_deg_body,
        grid=(B,),
        in_specs=[pl.BlockSpec((1, 8, NP), lambda g: (g, 0, 0))],
        out_specs=[
            pl.BlockSpec((1, 1, NP), lambda g: (g, 0, 0)),
            pl.BlockSpec((1, 1, NP), lambda g: (g, 0, 0)),
        ],
        out_shape=[
            jax.ShapeDtypeStruct((B, 1, NP), jnp.float32),
            jax.ShapeDtypeStruct((B, 1, NP), jnp.float32),
        ],
    )(part)


_CH_C = 2000


@functools.partial(
    pl.kernel,
    out_type=(
        jax.ShapeDtypeStruct((E,), jnp.float32),
        jax.ShapeDtypeStruct((E,), jnp.int32),
        jax.ShapeDtypeStruct((NTILES * NP,), jnp.float32),
    ),
    scratch_types=[
        pltpu.VMEM((NP_PAD,), jnp.float32),
        pltpu.VMEM((NP_PAD,), jnp.float32),
        pltpu.VMEM((NP_PAD,), jnp.float32),
        pltpu.VMEM((_CH_C,), jnp.int32),
        pltpu.VMEM((_CH_C,), jnp.int32),
        pltpu.VMEM((_CH_C,), jnp.float32),
        pltpu.VMEM((_CH_C,), jnp.float32),
        pltpu.VMEM((_CH_C,), jnp.int32),
    ],
    mesh=_MESH,
    compiler_params=pltpu.CompilerParams(needs_layout_passes=False),
)
def _norm_kernel(ei_hbm, ew_hbm, is_hbm, x_hbm, norm_hbm, pk_hbm, agg0_hbm,
                 is_loc, x_loc, agg0_loc, sbuf, dbuf, wbuf, nbuf, pkbuf):
    t = _wid()
    g = t // 8
    base = g * NP
    e0 = t * EPT
    pltpu.sync_copy(is_hbm.at[pl.ds(base, NP)], is_loc.at[pl.ds(0, NP)])
    pltpu.sync_copy(x_hbm.at[pl.ds(base, NP)], x_loc.at[pl.ds(0, NP)])
    _zero_f32(agg0_loc, NP_PAD)

    def chunk(c, _):
        off = e0 + c * _CH_C
        pltpu.sync_copy(ei_hbm.at[pl.ds(off, _CH_C)], sbuf)
        pltpu.sync_copy(ei_hbm.at[pl.ds(E + off, _CH_C)], dbuf)
        pltpu.sync_copy(ew_hbm.at[pl.ds(off, _CH_C)], wbuf)

        @plsc.parallel_loop(0, _CH_C // 16, unroll=8)
        def inner(i):
            sl = pl.ds(i * 16, 16)
            s = sbuf[sl] - base
            d = dbuf[sl] - base
            pkbuf[sl] = s | (d << 16)
            a = plsc.load_gather(is_loc, [s])
            b = plsc.load_gather(is_loc, [d])
            nv = wbuf[sl] * a * b
            nbuf[sl] = nv
            xv = plsc.load_gather(x_loc, [s])
            plsc.addupdate_scatter(agg0_loc, [d], nv * xv)
        pltpu.sync_copy(nbuf, norm_hbm.at[pl.ds(off, _CH_C)])
        pltpu.sync_copy(pkbuf, pk_hbm.at[pl.ds(off, _CH_C)])
        return 0

    lax.fori_loop(0, EPT // _CH_C, chunk, 0)
    pltpu.sync_copy(agg0_loc.at[pl.ds(0, NP)],
                    agg0_hbm.at[pl.ds(t * NP, NP)])


def _layer0_body(part_ref, x_ref, sc_ref, w0_ref, b0_ref, out_ref):
    agg = (jnp.sum(part_ref[...], axis=1, keepdims=True)
           + sc_ref[...] * x_ref[...])
    h = w0_ref[...] * agg + b0_ref[...]
    out_ref[...] = jnp.maximum(h, 0.0)


def _layer0(agg0_part, x3, selfc3, w0col, b0col):
    part = agg0_part.reshape(B, 8, NP)
    return pl.pallas_call(
        _layer0_body,
        grid=(B,),
        in_specs=[
            pl.BlockSpec((1, 8, NP), lambda g: (g, 0, 0)),
            pl.BlockSpec((1, 1, NP), lambda g: (g, 0, 0)),
            pl.BlockSpec((1, 1, NP), lambda g: (g, 0, 0)),
            pl.BlockSpec((F, 1), lambda g: (0, 0)),
            pl.BlockSpec((F, 1), lambda g: (0, 0)),
        ],
        out_specs=pl.BlockSpec((1, F, NP), lambda g: (g, 0, 0)),
        out_shape=jax.ShapeDtypeStruct((B, F, NP), jnp.float32),
    )(part, x3, selfc3, w0col, b0col)


_CH_E = 4000
_NCH_E = EP // _CH_E
_NPAIR = _NCH_E // 2


@functools.partial(
    pl.kernel,
    out_type=jax.ShapeDtypeStruct((B * F * NP,), jnp.float32),
    scratch_types=[
        pltpu.VMEM((NP_PAD,), jnp.float32),
        pltpu.VMEM((NP_PAD,), jnp.float32),
        pltpu.VMEM((NP_PAD,), jnp.float32),
        pltpu.VMEM((NP_PAD,), jnp.float32),
        pltpu.VMEM((_CH_E,), jnp.int32),
        pltpu.VMEM((_CH_E,), jnp.float32),
        pltpu.VMEM((_CH_E,), jnp.int32),
        pltpu.VMEM((_CH_E,), jnp.float32),
        pltpu.SemaphoreType.DMA,
        pltpu.SemaphoreType.DMA,
    ],
    mesh=_MESH,
    compiler_params=pltpu.CompilerParams(needs_layout_passes=False),
)
def _agg_kernel(ht_hbm, pk_hbm, nrm_hbm, out_hbm,
                h0_loc, h1_loc, a0_loc, a1_loc,
                pb0, nb0, pb1, nb1, sem0, sem1):
    t = _wid()
    g = t // 8
    p = t % 8
    r0 = (g * F + 2 * p) * NP
    pltpu.sync_copy(ht_hbm.at[pl.ds(r0, NP)], h0_loc.at[pl.ds(0, NP)])
    pltpu.sync_copy(ht_hbm.at[pl.ds(r0 + NP, NP)], h1_loc.at[pl.ds(0, NP)])
    _zero_f32(a0_loc, NP_PAD)
    _zero_f32(a1_loc, NP_PAD)

    def start(c, pb, nb, sem):
        off = g * EP + c * _CH_E
        pltpu.make_async_copy(pk_hbm.at[pl.ds(off, _CH_E)], pb, sem).start()
        pltpu.make_async_copy(nrm_hbm.at[pl.ds(off, _CH_E)], nb, sem).start()

    def wait(pb, nb, sem):
        pltpu.make_async_copy(pk_hbm.at[pl.ds(0, _CH_E)], pb, sem).wait()
        pltpu.make_async_copy(nrm_hbm.at[pl.ds(0, _CH_E)], nb, sem).wait()

    def edge_pass(pb, nb):
        @plsc.parallel_loop(0, _CH_E // 16, unroll=16)
        def inner(i):
            sl = pl.ds(i * 16, 16)
            pk = pb[sl]
            s = pk & 0xFFFF
            d = pk >> 16
            nv = nb[sl]
            m0 = plsc.load_gather(h0_loc, [s]) * nv
            m1 = plsc.load_gather(h1_loc, [s]) * nv
            plsc.addupdate_scatter(a0_loc, [d], m0)
            plsc.addupdate_scatter(a1_loc, [d], m1)

    start(0, pb0, nb0, sem0)

    def pair(c2, _):
        c = 2 * c2
        start(c + 1, pb1, nb1, sem1)
        wait(pb0, nb0, sem0)
        edge_pass(pb0, nb0)

        @pl.when(c2 + 1 < _NPAIR)
        def _():
            start(c + 2, pb0, nb0, sem0)

        wait(pb1, nb1, sem1)
        edge_pass(pb1, nb1)
        return 0

    lax.fori_loop(0, _NPAIR, pair, 0)
    pltpu.sync_copy(a0_loc.at[pl.ds(0, NP)], out_hbm.at[pl.ds(r0, NP)])
    pltpu.sync_copy(a1_loc.at[pl.ds(0, NP)], out_hbm.at[pl.ds(r0 + NP, NP)])


def _mm_body(agg_ref, ht_ref, sc_ref, wt_ref, b_ref, out_ref, *, relu):
    a2 = agg_ref[...].reshape(F, NP)
    h2 = ht_ref[...].reshape(F, NP)
    sc2 = sc_ref[...].reshape(1, NP)
    tmp = a2 + sc2 * h2
    h = jnp.dot(wt_ref[...], tmp, preferred_element_type=jnp.float32)
    h = h + b_ref[...]
    if relu:
        h = jnp.maximum(h, 0.0)
    out_ref[...] = h[None]


def _matmul(aggT, hT, selfc3, wt, bcol, relu):
    return pl.pallas_call(
        functools.partial(_mm_body, relu=relu),
        grid=(B,),
        in_specs=[
            pl.BlockSpec((1, F, NP), lambda g: (g, 0, 0)),
            pl.BlockSpec((1, F, NP), lambda g: (g, 0, 0)),
            pl.BlockSpec((1, 1, NP), lambda g: (g, 0, 0)),
            pl.BlockSpec((F, F), lambda g: (0, 0)),
            pl.BlockSpec((F, 1), lambda g: (0, 0)),
        ],
        out_specs=pl.BlockSpec((1, F, NP), lambda g: (g, 0, 0)),
        out_shape=jax.ShapeDtypeStruct((B, F, NP), jnp.float32),
    )(aggT, hT, selfc3, wt, bcol)


def _gsum_body(ht_ref, out_ref):
    out_ref[...] = jnp.sum(ht_ref[...], axis=2, keepdims=True)


def _gsum(hT3):
    return pl.pallas_call(
        _gsum_body,
        grid=(B,),
        in_specs=[pl.BlockSpec((1, F, NP), lambda g: (g, 0, 0))],
        out_specs=pl.BlockSpec((1, F, 1), lambda g: (g, 0, 0)),
        out_shape=jax.ShapeDtypeStruct((B, F, 1), jnp.float32),
    )(hT3)


def _pool_body(ht_ref, gs_ref, watt_ref, out_ref):
    g = pl.program_id(0)
    gmean = gs_ref[...].reshape(B, F) * (1.0 / NP)
    ctx = jnp.tanh(jnp.dot(gmean, watt_ref[...],
                           preferred_element_type=jnp.float32))
    sel = (lax.broadcasted_iota(jnp.int32, (B, 1), 0) == g).astype(jnp.float32)
    cg = jnp.sum(ctx * sel, axis=0, keepdims=True).reshape(F, 1)
    h = ht_ref[...].reshape(F, NP)
    scores = jax.nn.sigmoid(jnp.sum(h * cg, axis=0, keepdims=True))
    gf = jnp.sum(h * scores, axis=1, keepdims=True)
    node = h.T
    gfeat = jnp.broadcast_to(gf.reshape(1, F), (NP, F))
    out_ref[...] = jnp.concatenate([node, gfeat], axis=1)[None]


def _pool(hT3, gsum, watt):
    return pl.pallas_call(
        _pool_body,
        grid=(B,),
        in_specs=[
            pl.BlockSpec((1, F, NP), lambda g: (g, 0, 0)),
            pl.BlockSpec((B, F, 1), lambda g: (0, 0, 0)),
            pl.BlockSpec((F, F), lambda g: (0, 0)),
        ],
        out_specs=pl.BlockSpec((1, NP, 2 * F), lambda g: (g, 0, 0)),
        out_shape=jax.ShapeDtypeStruct((B, NP, 2 * F), jnp.float32),
    )(hT3, gsum, watt)


def kernel(x, edge_index, edge_weight, batch, W0, b0, Wh, bh, W_att):
    assert x.shape == (N, 1) and edge_index.shape == (2, E)
    LH = Wh.shape[0]

    ei_flat = edge_index.reshape(2 * E)
    x_flat = x.reshape(N)
    x3 = x.reshape(B, 1, NP)

    deg_part = _deg_kernel(ei_flat, edge_weight)
    is3, selfc3 = _merge_deg(deg_part)
    is_flat = is3.reshape(N)

    norm, pk, agg0_part = _norm_kernel(ei_flat, edge_weight, is_flat, x_flat)

    w0col = W0.reshape(F, 1)
    b0col = b0.reshape(F, 1)
    hT3 = _layer0(agg0_part, x3, selfc3, w0col, b0col)

    for i in range(LH):
        aggT_flat = _agg_kernel(hT3.reshape(B * F * NP), pk, norm)
        aggT3 = aggT_flat.reshape(B, F, NP)
        wt = Wh[i].T
        bcol = bh[i].reshape(F, 1)
        hT3 = _matmul(aggT3, hT3, selfc3, wt, bcol, relu=(i < LH - 1))

    gsum = _gsum(hT3)
    state = _pool(hT3, gsum, W_att)
    return state


# speedup vs baseline: 53.9101x; 1.0418x over previous
import functools

import jax
import jax.numpy as jnp
from jax import lax
from jax.experimental import pallas as pl
from jax.experimental.pallas import tpu as pltpu
from jax.experimental.pallas import tpu_sc as plsc

N = 100000
B = 4
NP = N // B
E = 1600000
EP = E // B
F = 16
NTILES = 32
EPT = E // NTILES
NP_PAD = 25008

_MESH = plsc.VectorSubcoreMesh(core_axis_name="c", subcore_axis_name="s")


def _wid():
    return lax.axis_index("c") * 16 + lax.axis_index("s")


def _zero_f32(ref, nwords):
    z = jnp.zeros((16,), jnp.float32)

    @plsc.parallel_loop(0, nwords // 16, unroll=8)
    def body(i):
        ref[pl.ds(i * 16, 16)] = z


_CH_A = 2000


_NCH_A = EPT // _CH_A


@functools.partial(
    pl.kernel,
    out_type=jax.ShapeDtypeStruct((NTILES * NP,), jnp.float32),
    scratch_types=[
        pltpu.VMEM((NP_PAD,), jnp.float32),
        pltpu.VMEM((_CH_A,), jnp.int32),
        pltpu.VMEM((_CH_A,), jnp.float32),
        pltpu.VMEM((_CH_A,), jnp.int32),
        pltpu.VMEM((_CH_A,), jnp.float32),
        pltpu.SemaphoreType.DMA,
        pltpu.SemaphoreType.DMA,
    ],
    mesh=_MESH,
    compiler_params=pltpu.CompilerParams(needs_layout_passes=False),
)
def _deg_kernel(ei_hbm, ew_hbm, out_hbm, deg_loc,
                db0, wb0, db1, wb1, sem0, sem1):
    t = _wid()
    g = t // 8
    base = g * NP
    e0 = t * EPT
    _zero_f32(deg_loc, NP_PAD)

    def start(c, db, wb, sem):
        off = e0 + c * _CH_A
        pltpu.make_async_copy(
            ei_hbm.at[pl.ds(E + off, _CH_A)], db, sem).start()
        pltpu.make_async_copy(ew_hbm.at[pl.ds(off, _CH_A)], wb, sem).start()

    def wait(db, wb, sem):
        pltpu.make_async_copy(ei_hbm.at[pl.ds(0, _CH_A)], db, sem).wait()
        pltpu.make_async_copy(ew_hbm.at[pl.ds(0, _CH_A)], wb, sem).wait()

    def edge_pass(db, wb):
        @plsc.parallel_loop(0, _CH_A // 16, unroll=16)
        def inner(i):
            sl = pl.ds(i * 16, 16)
            d = db[sl] - base
            w = wb[sl]
            plsc.addupdate_scatter(deg_loc, [d], w)

    start(0, db0, wb0, sem0)

    def pair(c2, _):
        c = 2 * c2
        start(c + 1, db1, wb1, sem1)
        wait(db0, wb0, sem0)
        edge_pass(db0, wb0)

        @pl.when(c + 2 < _NCH_A)
        def _():
            start(c + 2, db0, wb0, sem0)

        wait(db1, wb1, sem1)
        edge_pass(db1, wb1)
        return 0

    lax.fori_loop(0, _NCH_A // 2, pair, 0)
    wait(db0, wb0, sem0)
    edge_pass(db0, wb0)
    pltpu.sync_copy(deg_loc.at[pl.ds(0, NP)], out_hbm.at[pl.ds(t * NP, NP)])


def _merge_deg_body(part_ref, is_ref, sc_ref):
    deg = jnp.sum(part_ref[...], axis=1, keepdims=True) + 1.0
    inv = lax.rsqrt(deg)
    is_ref[...] = inv
    sc_ref[...] = inv * inv


def _merge_deg(deg_part):
    part = deg_part.reshape(B, 8, NP)
    return pl.pallas_call(
        _merge_deg_body,
        grid=(B,),
        in_specs=[pl.BlockSpec((1, 8, NP), lambda g: (g, 0, 0))],
        out_specs=[
            pl.BlockSpec((1, 1, NP), lambda g: (g, 0, 0)),
            pl.BlockSpec((1, 1, NP), lambda g: (g, 0, 0)),
        ],
        out_shape=[
            jax.ShapeDtypeStruct((B, 1, NP), jnp.float32),
            jax.ShapeDtypeStruct((B, 1, NP), jnp.float32),
        ],
    )(part)


_CH_C = 2000


_NCH_C = EPT // _CH_C


@functools.partial(
    pl.kernel,
    out_type=(
        jax.ShapeDtypeStruct((E,), jnp.float32),
        jax.ShapeDtypeStruct((E,), jnp.int32),
        jax.ShapeDtypeStruct((NTILES * NP,), jnp.float32),
    ),
    scratch_types=[
        pltpu.VMEM((NP_PAD,), jnp.float32),
        pltpu.VMEM((NP_PAD,), jnp.float32),
        pltpu.VMEM((NP_PAD,), jnp.float32),
        pltpu.VMEM((_CH_C,), jnp.int32),
        pltpu.VMEM((_CH_C,), jnp.int32),
        pltpu.VMEM((_CH_C,), jnp.float32),
        pltpu.VMEM((_CH_C,), jnp.int32),
        pltpu.VMEM((_CH_C,), jnp.int32),
        pltpu.VMEM((_CH_C,), jnp.float32),
        pltpu.VMEM((_CH_C,), jnp.float32),
        pltpu.VMEM((_CH_C,), jnp.int32),
        pltpu.VMEM((_CH_C,), jnp.float32),
        pltpu.VMEM((_CH_C,), jnp.int32),
        pltpu.SemaphoreType.DMA,
        pltpu.SemaphoreType.DMA,
        pltpu.SemaphoreType.DMA,
        pltpu.SemaphoreType.DMA,
    ],
    mesh=_MESH,
    compiler_params=pltpu.CompilerParams(needs_layout_passes=False),
)
def _norm_kernel(ei_hbm, ew_hbm, is_hbm, x_hbm, norm_hbm, pk_hbm, agg0_hbm,
                 is_loc, x_loc, agg0_loc,
                 sb0, db0, wb0, sb1, db1, wb1,
                 nb0, pkb0, nb1, pkb1,
                 semi0, semi1, semo0, semo1):
    t = _wid()
    g = t // 8
    base = g * NP
    e0 = t * EPT
    pltpu.sync_copy(is_hbm.at[pl.ds(base, NP)], is_loc.at[pl.ds(0, NP)])
    pltpu.sync_copy(x_hbm.at[pl.ds(base, NP)], x_loc.at[pl.ds(0, NP)])
    _zero_f32(agg0_loc, NP_PAD)

    def start_in(c, sb, db, wb, sem):
        off = e0 + c * _CH_C
        pltpu.make_async_copy(ei_hbm.at[pl.ds(off, _CH_C)], sb, sem).start()
        pltpu.make_async_copy(
            ei_hbm.at[pl.ds(E + off, _CH_C)], db, sem).start()
        pltpu.make_async_copy(ew_hbm.at[pl.ds(off, _CH_C)], wb, sem).start()

    def wait_in(sb, db, wb, sem):
        pltpu.make_async_copy(ei_hbm.at[pl.ds(0, _CH_C)], sb, sem).wait()
        pltpu.make_async_copy(ei_hbm.at[pl.ds(0, _CH_C)], db, sem).wait()
        pltpu.make_async_copy(ew_hbm.at[pl.ds(0, _CH_C)], wb, sem).wait()

    def start_out(c, nb, pkb, sem):
        off = e0 + c * _CH_C
        pltpu.make_async_copy(nb, norm_hbm.at[pl.ds(off, _CH_C)], sem).start()
        pltpu.make_async_copy(pkb, pk_hbm.at[pl.ds(off, _CH_C)], sem).start()

    def wait_out(nb, pkb, sem):
        pltpu.make_async_copy(nb, norm_hbm.at[pl.ds(0, _CH_C)], sem).wait()
        pltpu.make_async_copy(pkb, pk_hbm.at[pl.ds(0, _CH_C)], sem).wait()

    def edge_pass(sb, db, wb, nb, pkb):
        @plsc.parallel_loop(0, _CH_C // 16, unroll=16)
        def inner(i):
            sl = pl.ds(i * 16, 16)
            s = sb[sl] - base
            d = db[sl] - base
            pkb[sl] = s | (d << 16)
            a = plsc.load_gather(is_loc, [s])
            b = plsc.load_gather(is_loc, [d])
            nv = wb[sl] * a * b
            nb[sl] = nv
            xv = plsc.load_gather(x_loc, [s])
            plsc.addupdate_scatter(agg0_loc, [d], nv * xv)

    start_in(0, sb0, db0, wb0, semi0)

    def pair(c2, _):
        c = 2 * c2
        start_in(c + 1, sb1, db1, wb1, semi1)
        wait_in(sb0, db0, wb0, semi0)

        @pl.when(c2 > 0)
        def _():
            wait_out(nb0, pkb0, semo0)

        edge_pass(sb0, db0, wb0, nb0, pkb0)
        start_out(c, nb0, pkb0, semo0)

        @pl.when(c + 2 < _NCH_C)
        def _():
            start_in(c + 2, sb0, db0, wb0, semi0)

        wait_in(sb1, db1, wb1, semi1)

        @pl.when(c2 > 0)
        def _():
            wait_out(nb1, pkb1, semo1)

        edge_pass(sb1, db1, wb1, nb1, pkb1)
        start_out(c + 1, nb1, pkb1, semo1)
        return 0

    lax.fori_loop(0, _NCH_C // 2, pair, 0)
    wait_in(sb0, db0, wb0, semi0)
    wait_out(nb0, pkb0, semo0)
    edge_pass(sb0, db0, wb0, nb0, pkb0)
    start_out(_NCH_C - 1, nb0, pkb0, semo0)
    wait_out(nb0, pkb0, semo0)
    wait_out(nb1, pkb1, semo1)
    pltpu.sync_copy(agg0_loc.at[pl.ds(0, NP)],
                    agg0_hbm.at[pl.ds(t * NP, NP)])


def _layer0_body(part_ref, x_ref, sc_ref, w0_ref, b0_ref, out_ref):
    agg = (jnp.sum(part_ref[...], axis=1, keepdims=True)
           + sc_ref[...] * x_ref[...])
    h = w0_ref[...] * agg + b0_ref[...]
    out_ref[...] = jnp.maximum(h, 0.0)


def _layer0(agg0_part, x3, selfc3, w0col, b0col):
    part = agg0_part.reshape(B, 8, NP)
    return pl.pallas_call(
        _layer0_body,
        grid=(B,),
        in_specs=[
            pl.BlockSpec((1, 8, NP), lambda g: (g, 0, 0)),
            pl.BlockSpec((1, 1, NP), lambda g: (g, 0, 0)),
            pl.BlockSpec((1, 1, NP), lambda g: (g, 0, 0)),
            pl.BlockSpec((F, 1), lambda g: (0, 0)),
            pl.BlockSpec((F, 1), lambda g: (0, 0)),
        ],
        out_specs=pl.BlockSpec((1, F, NP), lambda g: (g, 0, 0)),
        out_shape=jax.ShapeDtypeStruct((B, F, NP), jnp.float32),
    )(part, x3, selfc3, w0col, b0col)


_CH_E = 4000
_NCH_E = EP // _CH_E
_NPAIR = _NCH_E // 2


@functools.partial(
    pl.kernel,
    out_type=jax.ShapeDtypeStruct((B * F * NP,), jnp.float32),
    scratch_types=[
        pltpu.VMEM((NP_PAD,), jnp.float32),
        pltpu.VMEM((NP_PAD,), jnp.float32),
        pltpu.VMEM((NP_PAD,), jnp.float32),
        pltpu.VMEM((NP_PAD,), jnp.float32),
        pltpu.VMEM((_CH_E,), jnp.int32),
        pltpu.VMEM((_CH_E,), jnp.float32),
        pltpu.VMEM((_CH_E,), jnp.int32),
        pltpu.VMEM((_CH_E,), jnp.float32),
        pltpu.SemaphoreType.DMA,
        pltpu.SemaphoreType.DMA,
    ],
    mesh=_MESH,
    compiler_params=pltpu.CompilerParams(needs_layout_passes=False),
)
def _agg_kernel(ht_hbm, pk_hbm, nrm_hbm, out_hbm,
                h0_loc, h1_loc, a0_loc, a1_loc,
                pb0, nb0, pb1, nb1, sem0, sem1):
    t = _wid()
    g = t // 8
    p = t % 8
    r0 = (g * F + 2 * p) * NP
    pltpu.sync_copy(ht_hbm.at[pl.ds(r0, NP)], h0_loc.at[pl.ds(0, NP)])
    pltpu.sync_copy(ht_hbm.at[pl.ds(r0 + NP, NP)], h1_loc.at[pl.ds(0, NP)])
    _zero_f32(a0_loc, NP_PAD)
    _zero_f32(a1_loc, NP_PAD)

    def start(c, pb, nb, sem):
        off = g * EP + c * _CH_E
        pltpu.make_async_copy(pk_hbm.at[pl.ds(off, _CH_E)], pb, sem).start()
        pltpu.make_async_copy(nrm_hbm.at[pl.ds(off, _CH_E)], nb, sem).start()

    def wait(pb, nb, sem):
        pltpu.make_async_copy(pk_hbm.at[pl.ds(0, _CH_E)], pb, sem).wait()
        pltpu.make_async_copy(nrm_hbm.at[pl.ds(0, _CH_E)], nb, sem).wait()

    def edge_pass(pb, nb):
        @plsc.parallel_loop(0, _CH_E // 16, unroll=16)
        def inner(i):
            sl = pl.ds(i * 16, 16)
            pk = pb[sl]
            s = pk & 0xFFFF
            d = pk >> 16
            nv = nb[sl]
            m0 = plsc.load_gather(h0_loc, [s]) * nv
            m1 = plsc.load_gather(h1_loc, [s]) * nv
            plsc.addupdate_scatter(a0_loc, [d], m0)
            plsc.addupdate_scatter(a1_loc, [d], m1)

    start(0, pb0, nb0, sem0)

    def pair(c2, _):
        c = 2 * c2
        start(c + 1, pb1, nb1, sem1)
        wait(pb0, nb0, sem0)
        edge_pass(pb0, nb0)

        @pl.when(c2 + 1 < _NPAIR)
        def _():
            start(c + 2, pb0, nb0, sem0)

        wait(pb1, nb1, sem1)
        edge_pass(pb1, nb1)
        return 0

    lax.fori_loop(0, _NPAIR, pair, 0)
    pltpu.sync_copy(a0_loc.at[pl.ds(0, NP)], out_hbm.at[pl.ds(r0, NP)])
    pltpu.sync_copy(a1_loc.at[pl.ds(0, NP)], out_hbm.at[pl.ds(r0 + NP, NP)])


def _mm_body(agg_ref, ht_ref, sc_ref, wt_ref, b_ref, out_ref, *, relu):
    a2 = agg_ref[...].reshape(F, NP)
    h2 = ht_ref[...].reshape(F, NP)
    sc2 = sc_ref[...].reshape(1, NP)
    tmp = a2 + sc2 * h2
    h = jnp.dot(wt_ref[...], tmp, preferred_element_type=jnp.float32)
    h = h + b_ref[...]
    if relu:
        h = jnp.maximum(h, 0.0)
    out_ref[...] = h[None]


def _matmul(aggT, hT, selfc3, wt, bcol, relu):
    return pl.pallas_call(
        functools.partial(_mm_body, relu=relu),
        grid=(B,),
        in_specs=[
            pl.BlockSpec((1, F, NP), lambda g: (g, 0, 0)),
            pl.BlockSpec((1, F, NP), lambda g: (g, 0, 0)),
            pl.BlockSpec((1, 1, NP), lambda g: (g, 0, 0)),
            pl.BlockSpec((F, F), lambda g: (0, 0)),
            pl.BlockSpec((F, 1), lambda g: (0, 0)),
        ],
        out_specs=pl.BlockSpec((1, F, NP), lambda g: (g, 0, 0)),
        out_shape=jax.ShapeDtypeStruct((B, F, NP), jnp.float32),
    )(aggT, hT, selfc3, wt, bcol)


def _mm_final_body(agg_ref, ht_ref, sc_ref, wt_ref, b_ref, out_ref, gs_ref):
    a2 = agg_ref[...].reshape(F, NP)
    h2 = ht_ref[...].reshape(F, NP)
    sc2 = sc_ref[...].reshape(1, NP)
    tmp = a2 + sc2 * h2
    h = jnp.dot(wt_ref[...], tmp, preferred_element_type=jnp.float32)
    h = h + b_ref[...]
    out_ref[...] = h[None]
    gs_ref[...] = jnp.sum(h, axis=1, keepdims=True)[None]


def _matmul_final(aggT, hT, selfc3, wt, bcol):
    return pl.pallas_call(
        _mm_final_body,
        grid=(B,),
        in_specs=[
            pl.BlockSpec((1, F, NP), lambda g: (g, 0, 0)),
            pl.BlockSpec((1, F, NP), lambda g: (g, 0, 0)),
            pl.BlockSpec((1, 1, NP), lambda g: (g, 0, 0)),
            pl.BlockSpec((F, F), lambda g: (0, 0)),
            pl.BlockSpec((F, 1), lambda g: (0, 0)),
        ],
        out_specs=[
            pl.BlockSpec((1, F, NP), lambda g: (g, 0, 0)),
            pl.BlockSpec((1, F, 1), lambda g: (g, 0, 0)),
        ],
        out_shape=[
            jax.ShapeDtypeStruct((B, F, NP), jnp.float32),
            jax.ShapeDtypeStruct((B, F, 1), jnp.float32),
        ],
    )(aggT, hT, selfc3, wt, bcol)


def _pool_body(ht_ref, gs_ref, watt_ref, out_ref):
    g = pl.program_id(0)
    gmean = gs_ref[...].reshape(B, F) * (1.0 / NP)
    ctx = jnp.tanh(jnp.dot(gmean, watt_ref[...],
                           preferred_element_type=jnp.float32))
    sel = (lax.broadcasted_iota(jnp.int32, (B, 1), 0) == g).astype(jnp.float32)
    cg = jnp.sum(ctx * sel, axis=0, keepdims=True).reshape(F, 1)
    h = ht_ref[...].reshape(F, NP)
    scores = jax.nn.sigmoid(jnp.sum(h * cg, axis=0, keepdims=True))
    gf = jnp.sum(h * scores, axis=1, keepdims=True)
    node = h.T
    gfeat = jnp.broadcast_to(gf.reshape(1, F), (NP, F))
    out_ref[...] = jnp.concatenate([node, gfeat], axis=1)[None]


def _pool(hT3, gsum, watt):
    return pl.pallas_call(
        _pool_body,
        grid=(B,),
        in_specs=[
            pl.BlockSpec((1, F, NP), lambda g: (g, 0, 0)),
            pl.BlockSpec((B, F, 1), lambda g: (0, 0, 0)),
            pl.BlockSpec((F, F), lambda g: (0, 0)),
        ],
        out_specs=pl.BlockSpec((1, NP, 2 * F), lambda g: (g, 0, 0)),
        out_shape=jax.ShapeDtypeStruct((B, NP, 2 * F), jnp.float32),
    )(hT3, gsum, watt)


def kernel(x, edge_index, edge_weight, batch, W0, b0, Wh, bh, W_att):
    assert x.shape == (N, 1) and edge_index.shape == (2, E)
    LH = Wh.shape[0]

    ei_flat = edge_index.reshape(2 * E)
    x_flat = x.reshape(N)
    x3 = x.reshape(B, 1, NP)

    deg_part = _deg_kernel(ei_flat, edge_weight)
    is3, selfc3 = _merge_deg(deg_part)
    is_flat = is3.reshape(N)

    norm, pk, agg0_part = _norm_kernel(ei_flat, edge_weight, is_flat, x_flat)

    w0col = W0.reshape(F, 1)
    b0col = b0.reshape(F, 1)
    hT3 = _layer0(agg0_part, x3, selfc3, w0col, b0col)

    for i in range(LH):
        aggT_flat = _agg_kernel(hT3.reshape(B * F * NP), pk, norm)
        aggT3 = aggT_flat.reshape(B, F, NP)
        wt = Wh[i].T
        bcol = bh[i].reshape(F, 1)
        if i < LH - 1:
            hT3 = _matmul(aggT3, hT3, selfc3, wt, bcol, relu=True)
        else:
            hT3, gsum = _matmul_final(aggT3, hT3, selfc3, wt, bcol)

    state = _pool(hT3, gsum, W_att)
    return state


# speedup vs baseline: 56.1396x; 1.0414x over previous
import functools

import jax
import jax.numpy as jnp
from jax import lax
from jax.experimental import pallas as pl
from jax.experimental.pallas import tpu as pltpu
from jax.experimental.pallas import tpu_sc as plsc

N = 100000
B = 4
NP = N // B
E = 1600000
EP = E // B
F = 16
NTILES = 32
EPT = E // NTILES
NP_PAD = 25008

_MESH = plsc.VectorSubcoreMesh(core_axis_name="c", subcore_axis_name="s")


def _wid():
    return lax.axis_index("c") * 16 + lax.axis_index("s")


def _zero_f32(ref, nwords):
    z = jnp.zeros((16,), jnp.float32)

    @plsc.parallel_loop(0, nwords // 16, unroll=8)
    def body(i):
        ref[pl.ds(i * 16, 16)] = z


_CH_A = 2000


_NCH_A = EPT // _CH_A


@functools.partial(
    pl.kernel,
    out_type=jax.ShapeDtypeStruct((NTILES * NP,), jnp.float32),
    scratch_types=[
        pltpu.VMEM((NP_PAD,), jnp.float32),
        pltpu.VMEM((_CH_A,), jnp.int32),
        pltpu.VMEM((_CH_A,), jnp.float32),
        pltpu.VMEM((_CH_A,), jnp.int32),
        pltpu.VMEM((_CH_A,), jnp.float32),
        pltpu.SemaphoreType.DMA,
        pltpu.SemaphoreType.DMA,
    ],
    mesh=_MESH,
    compiler_params=pltpu.CompilerParams(needs_layout_passes=False),
)
def _deg_kernel(ei_hbm, ew_hbm, out_hbm, deg_loc,
                db0, wb0, db1, wb1, sem0, sem1):
    t = _wid()
    g = t // 8
    base = g * NP
    e0 = t * EPT
    _zero_f32(deg_loc, NP_PAD)

    def start(c, db, wb, sem):
        off = e0 + c * _CH_A
        pltpu.make_async_copy(
            ei_hbm.at[pl.ds(E + off, _CH_A)], db, sem).start()
        pltpu.make_async_copy(ew_hbm.at[pl.ds(off, _CH_A)], wb, sem).start()

    def wait(db, wb, sem):
        pltpu.make_async_copy(ei_hbm.at[pl.ds(0, _CH_A)], db, sem).wait()
        pltpu.make_async_copy(ew_hbm.at[pl.ds(0, _CH_A)], wb, sem).wait()

    def edge_pass(db, wb):
        @plsc.parallel_loop(0, _CH_A // 16, unroll=16)
        def inner(i):
            sl = pl.ds(i * 16, 16)
            d = db[sl] - base
            w = wb[sl]
            plsc.addupdate_scatter(deg_loc, [d], w)

    start(0, db0, wb0, sem0)

    def pair(c2, _):
        c = 2 * c2
        start(c + 1, db1, wb1, sem1)
        wait(db0, wb0, sem0)
        edge_pass(db0, wb0)

        @pl.when(c + 2 < _NCH_A)
        def _():
            start(c + 2, db0, wb0, sem0)

        wait(db1, wb1, sem1)
        edge_pass(db1, wb1)
        return 0

    lax.fori_loop(0, _NCH_A // 2, pair, 0)
    wait(db0, wb0, sem0)
    edge_pass(db0, wb0)
    pltpu.sync_copy(deg_loc.at[pl.ds(0, NP)], out_hbm.at[pl.ds(t * NP, NP)])


def _merge_deg_body(part_ref, is_ref, sc_ref):
    deg = jnp.sum(part_ref[...], axis=1, keepdims=True) + 1.0
    inv = lax.rsqrt(deg)
    is_ref[...] = inv
    sc_ref[...] = inv * inv


def _merge_deg(deg_part):
    part = deg_part.reshape(B, 8, NP)
    return pl.pallas_call(
        _merge_deg_body,
        grid=(B,),
        in_specs=[pl.BlockSpec((1, 8, NP), lambda g: (g, 0, 0))],
        out_specs=[
            pl.BlockSpec((1, 1, NP), lambda g: (g, 0, 0)),
            pl.BlockSpec((1, 1, NP), lambda g: (g, 0, 0)),
        ],
        out_shape=[
            jax.ShapeDtypeStruct((B, 1, NP), jnp.float32),
            jax.ShapeDtypeStruct((B, 1, NP), jnp.float32),
        ],
    )(part)


_CH_C = 2000


_NCH_C = EPT // _CH_C


@functools.partial(
    pl.kernel,
    out_type=(
        jax.ShapeDtypeStruct((E,), jnp.float32),
        jax.ShapeDtypeStruct((E,), jnp.int32),
        jax.ShapeDtypeStruct((NTILES * NP,), jnp.float32),
    ),
    scratch_types=[
        pltpu.VMEM((NP_PAD,), jnp.float32),
        pltpu.VMEM((NP_PAD,), jnp.float32),
        pltpu.VMEM((NP_PAD,), jnp.float32),
        pltpu.VMEM((_CH_C,), jnp.int32),
        pltpu.VMEM((_CH_C,), jnp.int32),
        pltpu.VMEM((_CH_C,), jnp.float32),
        pltpu.VMEM((_CH_C,), jnp.int32),
        pltpu.VMEM((_CH_C,), jnp.int32),
        pltpu.VMEM((_CH_C,), jnp.float32),
        pltpu.VMEM((_CH_C,), jnp.float32),
        pltpu.VMEM((_CH_C,), jnp.int32),
        pltpu.VMEM((_CH_C,), jnp.float32),
        pltpu.VMEM((_CH_C,), jnp.int32),
        pltpu.SemaphoreType.DMA,
        pltpu.SemaphoreType.DMA,
        pltpu.SemaphoreType.DMA,
        pltpu.SemaphoreType.DMA,
    ],
    mesh=_MESH,
    compiler_params=pltpu.CompilerParams(needs_layout_passes=False),
)
def _norm_kernel(ei_hbm, ew_hbm, is_hbm, x_hbm, norm_hbm, pk_hbm, agg0_hbm,
                 is_loc, x_loc, agg0_loc,
                 sb0, db0, wb0, sb1, db1, wb1,
                 nb0, pkb0, nb1, pkb1,
                 semi0, semi1, semo0, semo1):
    t = _wid()
    g = t // 8
    base = g * NP
    e0 = t * EPT
    pltpu.sync_copy(is_hbm.at[pl.ds(base, NP)], is_loc.at[pl.ds(0, NP)])
    pltpu.sync_copy(x_hbm.at[pl.ds(base, NP)], x_loc.at[pl.ds(0, NP)])
    _zero_f32(agg0_loc, NP_PAD)

    def start_in(c, sb, db, wb, sem):
        off = e0 + c * _CH_C
        pltpu.make_async_copy(ei_hbm.at[pl.ds(off, _CH_C)], sb, sem).start()
        pltpu.make_async_copy(
            ei_hbm.at[pl.ds(E + off, _CH_C)], db, sem).start()
        pltpu.make_async_copy(ew_hbm.at[pl.ds(off, _CH_C)], wb, sem).start()

    def wait_in(sb, db, wb, sem):
        pltpu.make_async_copy(ei_hbm.at[pl.ds(0, _CH_C)], sb, sem).wait()
        pltpu.make_async_copy(ei_hbm.at[pl.ds(0, _CH_C)], db, sem).wait()
        pltpu.make_async_copy(ew_hbm.at[pl.ds(0, _CH_C)], wb, sem).wait()

    def start_out(c, nb, pkb, sem):
        off = e0 + c * _CH_C
        pltpu.make_async_copy(nb, norm_hbm.at[pl.ds(off, _CH_C)], sem).start()
        pltpu.make_async_copy(pkb, pk_hbm.at[pl.ds(off, _CH_C)], sem).start()

    def wait_out(nb, pkb, sem):
        pltpu.make_async_copy(nb, norm_hbm.at[pl.ds(0, _CH_C)], sem).wait()
        pltpu.make_async_copy(pkb, pk_hbm.at[pl.ds(0, _CH_C)], sem).wait()

    def edge_pass(sb, db, wb, nb, pkb):
        @plsc.parallel_loop(0, _CH_C // 16, unroll=16)
        def inner(i):
            sl = pl.ds(i * 16, 16)
            s = sb[sl] - base
            d = db[sl] - base
            pkb[sl] = s | (d << 16)
            a = plsc.load_gather(is_loc, [s])
            b = plsc.load_gather(is_loc, [d])
            nv = wb[sl] * a * b
            nb[sl] = nv
            xv = plsc.load_gather(x_loc, [s])
            plsc.addupdate_scatter(agg0_loc, [d], nv * xv)

    start_in(0, sb0, db0, wb0, semi0)

    def pair(c2, _):
        c = 2 * c2
        start_in(c + 1, sb1, db1, wb1, semi1)
        wait_in(sb0, db0, wb0, semi0)

        @pl.when(c2 > 0)
        def _():
            wait_out(nb0, pkb0, semo0)

        edge_pass(sb0, db0, wb0, nb0, pkb0)
        start_out(c, nb0, pkb0, semo0)

        @pl.when(c + 2 < _NCH_C)
        def _():
            start_in(c + 2, sb0, db0, wb0, semi0)

        wait_in(sb1, db1, wb1, semi1)

        @pl.when(c2 > 0)
        def _():
            wait_out(nb1, pkb1, semo1)

        edge_pass(sb1, db1, wb1, nb1, pkb1)
        start_out(c + 1, nb1, pkb1, semo1)
        return 0

    lax.fori_loop(0, _NCH_C // 2, pair, 0)
    wait_in(sb0, db0, wb0, semi0)
    wait_out(nb0, pkb0, semo0)
    edge_pass(sb0, db0, wb0, nb0, pkb0)
    start_out(_NCH_C - 1, nb0, pkb0, semo0)
    wait_out(nb0, pkb0, semo0)
    wait_out(nb1, pkb1, semo1)
    pltpu.sync_copy(agg0_loc.at[pl.ds(0, NP)],
                    agg0_hbm.at[pl.ds(t * NP, NP)])


def _layer0_body(part_ref, x_ref, sc_ref, w0_ref, b0_ref, out_ref):
    agg = (jnp.sum(part_ref[...], axis=1, keepdims=True)
           + sc_ref[...] * x_ref[...])
    h = w0_ref[...] * agg + b0_ref[...]
    out_ref[...] = jnp.maximum(h, 0.0)


def _layer0(agg0_part, x3, selfc3, w0col, b0col):
    part = agg0_part.reshape(B, 8, NP)
    return pl.pallas_call(
        _layer0_body,
        grid=(B,),
        in_specs=[
            pl.BlockSpec((1, 8, NP), lambda g: (g, 0, 0)),
            pl.BlockSpec((1, 1, NP), lambda g: (g, 0, 0)),
            pl.BlockSpec((1, 1, NP), lambda g: (g, 0, 0)),
            pl.BlockSpec((F, 1), lambda g: (0, 0)),
            pl.BlockSpec((F, 1), lambda g: (0, 0)),
        ],
        out_specs=pl.BlockSpec((1, F, NP), lambda g: (g, 0, 0)),
        out_shape=jax.ShapeDtypeStruct((B, F, NP), jnp.float32),
    )(part, x3, selfc3, w0col, b0col)


_CH_E = 4000
_NCH_E = EP // _CH_E
_NPAIR = _NCH_E // 2


@functools.partial(
    pl.kernel,
    out_type=jax.ShapeDtypeStruct((B * F * NP,), jnp.float32),
    scratch_types=[
        pltpu.VMEM((NP_PAD,), jnp.float32),
        pltpu.VMEM((NP_PAD,), jnp.float32),
        pltpu.VMEM((NP_PAD,), jnp.float32),
        pltpu.VMEM((NP_PAD,), jnp.float32),
        pltpu.VMEM((_CH_E,), jnp.int32),
        pltpu.VMEM((_CH_E,), jnp.float32),
        pltpu.VMEM((_CH_E,), jnp.int32),
        pltpu.VMEM((_CH_E,), jnp.float32),
        pltpu.SemaphoreType.DMA,
        pltpu.SemaphoreType.DMA,
    ],
    mesh=_MESH,
    compiler_params=pltpu.CompilerParams(needs_layout_passes=False),
)
def _agg_kernel(ht_hbm, pk_hbm, nrm_hbm, out_hbm,
                h0_loc, h1_loc, a0_loc, a1_loc,
                pb0, nb0, pb1, nb1, sem0, sem1):
    t = _wid()
    g = t // 8
    p = t % 8
    r0 = (g * F + 2 * p) * NP
    pltpu.sync_copy(ht_hbm.at[pl.ds(r0, NP)], h0_loc.at[pl.ds(0, NP)])
    pltpu.sync_copy(ht_hbm.at[pl.ds(r0 + NP, NP)], h1_loc.at[pl.ds(0, NP)])
    _zero_f32(a0_loc, NP_PAD)
    _zero_f32(a1_loc, NP_PAD)

    def start(c, pb, nb, sem):
        off = g * EP + c * _CH_E
        pltpu.make_async_copy(pk_hbm.at[pl.ds(off, _CH_E)], pb, sem).start()
        pltpu.make_async_copy(nrm_hbm.at[pl.ds(off, _CH_E)], nb, sem).start()

    def wait(pb, nb, sem):
        pltpu.make_async_copy(pk_hbm.at[pl.ds(0, _CH_E)], pb, sem).wait()
        pltpu.make_async_copy(nrm_hbm.at[pl.ds(0, _CH_E)], nb, sem).wait()

    def edge_pass(pb, nb):
        @plsc.parallel_loop(0, _CH_E // 16, unroll=8)
        def inner(i):
            sl = pl.ds(i * 16, 16)
            pk = pb[sl]
            s = pk & 0xFFFF
            d = pk >> 16
            nv = nb[sl]
            m0 = plsc.load_gather(h0_loc, [s]) * nv
            m1 = plsc.load_gather(h1_loc, [s]) * nv
            plsc.addupdate_scatter(a0_loc, [d], m0)
            plsc.addupdate_scatter(a1_loc, [d], m1)

    start(0, pb0, nb0, sem0)

    def pair(c2, _):
        c = 2 * c2
        start(c + 1, pb1, nb1, sem1)
        wait(pb0, nb0, sem0)
        edge_pass(pb0, nb0)

        @pl.when(c2 + 1 < _NPAIR)
        def _():
            start(c + 2, pb0, nb0, sem0)

        wait(pb1, nb1, sem1)
        edge_pass(pb1, nb1)
        return 0

    lax.fori_loop(0, _NPAIR, pair, 0)
    pltpu.sync_copy(a0_loc.at[pl.ds(0, NP)], out_hbm.at[pl.ds(r0, NP)])
    pltpu.sync_copy(a1_loc.at[pl.ds(0, NP)], out_hbm.at[pl.ds(r0 + NP, NP)])


def _mm_body(agg_ref, ht_ref, sc_ref, wt_ref, b_ref, out_ref, *, relu):
    a2 = agg_ref[...].reshape(F, NP)
    h2 = ht_ref[...].reshape(F, NP)
    sc2 = sc_ref[...].reshape(1, NP)
    tmp = a2 + sc2 * h2
    h = jnp.dot(wt_ref[...], tmp, preferred_element_type=jnp.float32)
    h = h + b_ref[...]
    if relu:
        h = jnp.maximum(h, 0.0)
    out_ref[...] = h[None]


def _matmul(aggT, hT, selfc3, wt, bcol, relu):
    return pl.pallas_call(
        functools.partial(_mm_body, relu=relu),
        grid=(B,),
        in_specs=[
            pl.BlockSpec((1, F, NP), lambda g: (g, 0, 0)),
            pl.BlockSpec((1, F, NP), lambda g: (g, 0, 0)),
            pl.BlockSpec((1, 1, NP), lambda g: (g, 0, 0)),
            pl.BlockSpec((F, F), lambda g: (0, 0)),
            pl.BlockSpec((F, 1), lambda g: (0, 0)),
        ],
        out_specs=pl.BlockSpec((1, F, NP), lambda g: (g, 0, 0)),
        out_shape=jax.ShapeDtypeStruct((B, F, NP), jnp.float32),
    )(aggT, hT, selfc3, wt, bcol)


def _mm_final_body(agg_ref, ht_ref, sc_ref, wt_ref, b_ref, out_ref, gs_ref):
    a2 = agg_ref[...].reshape(F, NP)
    h2 = ht_ref[...].reshape(F, NP)
    sc2 = sc_ref[...].reshape(1, NP)
    tmp = a2 + sc2 * h2
    h = jnp.dot(wt_ref[...], tmp, preferred_element_type=jnp.float32)
    h = h + b_ref[...]
    out_ref[...] = h[None]
    gs_ref[...] = jnp.sum(h, axis=1, keepdims=True)[None]


def _matmul_final(aggT, hT, selfc3, wt, bcol):
    return pl.pallas_call(
        _mm_final_body,
        grid=(B,),
        in_specs=[
            pl.BlockSpec((1, F, NP), lambda g: (g, 0, 0)),
            pl.BlockSpec((1, F, NP), lambda g: (g, 0, 0)),
            pl.BlockSpec((1, 1, NP), lambda g: (g, 0, 0)),
            pl.BlockSpec((F, F), lambda g: (0, 0)),
            pl.BlockSpec((F, 1), lambda g: (0, 0)),
        ],
        out_specs=[
            pl.BlockSpec((1, F, NP), lambda g: (g, 0, 0)),
            pl.BlockSpec((1, F, 1), lambda g: (g, 0, 0)),
        ],
        out_shape=[
            jax.ShapeDtypeStruct((B, F, NP), jnp.float32),
            jax.ShapeDtypeStruct((B, F, 1), jnp.float32),
        ],
    )(aggT, hT, selfc3, wt, bcol)


def _pool_body(ht_ref, gs_ref, watt_ref, out_ref):
    g = pl.program_id(0)
    gmean = gs_ref[...].reshape(B, F) * (1.0 / NP)
    ctx = jnp.tanh(jnp.dot(gmean, watt_ref[...],
                           preferred_element_type=jnp.float32))
    sel = (lax.broadcasted_iota(jnp.int32, (B, 1), 0) == g).astype(jnp.float32)
    cg = jnp.sum(ctx * sel, axis=0, keepdims=True).reshape(F, 1)
    h = ht_ref[...].reshape(F, NP)
    scores = jax.nn.sigmoid(jnp.sum(h * cg, axis=0, keepdims=True))
    gf = jnp.sum(h * scores, axis=1, keepdims=True)
    node = h.T
    gfeat = jnp.broadcast_to(gf.reshape(1, F), (NP, F))
    out_ref[...] = jnp.concatenate([node, gfeat], axis=1)[None]


def _pool(hT3, gsum, watt):
    return pl.pallas_call(
        _pool_body,
        grid=(B,),
        in_specs=[
            pl.BlockSpec((1, F, NP), lambda g: (g, 0, 0)),
            pl.BlockSpec((B, F, 1), lambda g: (0, 0, 0)),
            pl.BlockSpec((F, F), lambda g: (0, 0)),
        ],
        out_specs=pl.BlockSpec((1, NP, 2 * F), lambda g: (g, 0, 0)),
        out_shape=jax.ShapeDtypeStruct((B, NP, 2 * F), jnp.float32),
    )(hT3, gsum, watt)


def kernel(x, edge_index, edge_weight, batch, W0, b0, Wh, bh, W_att):
    assert x.shape == (N, 1) and edge_index.shape == (2, E)
    LH = Wh.shape[0]

    ei_flat = edge_index.reshape(2 * E)
    x_flat = x.reshape(N)
    x3 = x.reshape(B, 1, NP)

    deg_part = _deg_kernel(ei_flat, edge_weight)
    is3, selfc3 = _merge_deg(deg_part)
    is_flat = is3.reshape(N)

    norm, pk, agg0_part = _norm_kernel(ei_flat, edge_weight, is_flat, x_flat)

    w0col = W0.reshape(F, 1)
    b0col = b0.reshape(F, 1)
    hT3 = _layer0(agg0_part, x3, selfc3, w0col, b0col)

    for i in range(LH):
        aggT_flat = _agg_kernel(hT3.reshape(B * F * NP), pk, norm)
        aggT3 = aggT_flat.reshape(B, F, NP)
        wt = Wh[i].T
        bcol = bh[i].reshape(F, 1)
        if i < LH - 1:
            hT3 = _matmul(aggT3, hT3, selfc3, wt, bcol, relu=True)
        else:
            hT3, gsum = _matmul_final(aggT3, hT3, selfc3, wt, bcol)

    state = _pool(hT3, gsum, W_att)
    return state


# speedup vs baseline: 57.2450x; 1.0197x over previous
import functools

import jax
import jax.numpy as jnp
from jax import lax
from jax.experimental import pallas as pl
from jax.experimental.pallas import tpu as pltpu
from jax.experimental.pallas import tpu_sc as plsc

N = 100000
B = 4
NP = N // B
E = 1600000
EP = E // B
F = 16
NTILES = 32
EPT = E // NTILES
NP_PAD = 25008

_MESH = plsc.VectorSubcoreMesh(core_axis_name="c", subcore_axis_name="s")


def _wid():
    return lax.axis_index("c") * 16 + lax.axis_index("s")


def _zero_f32(ref, nwords):
    z = jnp.zeros((16,), jnp.float32)

    @plsc.parallel_loop(0, nwords // 16, unroll=8)
    def body(i):
        ref[pl.ds(i * 16, 16)] = z


_CH_A = 2000


_NCH_A = EPT // _CH_A


@functools.partial(
    pl.kernel,
    out_type=jax.ShapeDtypeStruct((NTILES * NP,), jnp.float32),
    scratch_types=[
        pltpu.VMEM((NP_PAD,), jnp.float32),
        pltpu.VMEM((_CH_A,), jnp.int32),
        pltpu.VMEM((_CH_A,), jnp.float32),
        pltpu.VMEM((_CH_A,), jnp.int32),
        pltpu.VMEM((_CH_A,), jnp.float32),
        pltpu.SemaphoreType.DMA,
        pltpu.SemaphoreType.DMA,
    ],
    mesh=_MESH,
    compiler_params=pltpu.CompilerParams(needs_layout_passes=False),
)
def _deg_kernel(ei_hbm, ew_hbm, out_hbm, deg_loc,
                db0, wb0, db1, wb1, sem0, sem1):
    t = _wid()
    g = t // 8
    base = g * NP
    e0 = t * EPT
    _zero_f32(deg_loc, NP_PAD)

    def start(c, db, wb, sem):
        off = e0 + c * _CH_A
        pltpu.make_async_copy(
            ei_hbm.at[pl.ds(E + off, _CH_A)], db, sem).start()
        pltpu.make_async_copy(ew_hbm.at[pl.ds(off, _CH_A)], wb, sem).start()

    def wait(db, wb, sem):
        pltpu.make_async_copy(ei_hbm.at[pl.ds(0, _CH_A)], db, sem).wait()
        pltpu.make_async_copy(ew_hbm.at[pl.ds(0, _CH_A)], wb, sem).wait()

    def edge_pass(db, wb):
        @plsc.parallel_loop(0, _CH_A // 16, unroll=16)
        def inner(i):
            sl = pl.ds(i * 16, 16)
            d = db[sl] - base
            w = wb[sl]
            plsc.addupdate_scatter(deg_loc, [d], w)

    start(0, db0, wb0, sem0)

    def pair(c2, _):
        c = 2 * c2
        start(c + 1, db1, wb1, sem1)
        wait(db0, wb0, sem0)
        edge_pass(db0, wb0)

        @pl.when(c + 2 < _NCH_A)
        def _():
            start(c + 2, db0, wb0, sem0)

        wait(db1, wb1, sem1)
        edge_pass(db1, wb1)
        return 0

    lax.fori_loop(0, _NCH_A // 2, pair, 0)
    wait(db0, wb0, sem0)
    edge_pass(db0, wb0)
    pltpu.sync_copy(deg_loc.at[pl.ds(0, NP)], out_hbm.at[pl.ds(t * NP, NP)])


def _merge_deg_body(part_ref, is_ref, sc_ref):
    deg = jnp.sum(part_ref[...], axis=1, keepdims=True) + 1.0
    inv = lax.rsqrt(deg)
    is_ref[...] = inv
    sc_ref[...] = inv * inv


def _merge_deg(deg_part):
    part = deg_part.reshape(B, 8, NP)
    return pl.pallas_call(
        _merge_deg_body,
        grid=(B,),
        in_specs=[pl.BlockSpec((1, 8, NP), lambda g: (g, 0, 0))],
        out_specs=[
            pl.BlockSpec((1, 1, NP), lambda g: (g, 0, 0)),
            pl.BlockSpec((1, 1, NP), lambda g: (g, 0, 0)),
        ],
        out_shape=[
            jax.ShapeDtypeStruct((B, 1, NP), jnp.float32),
            jax.ShapeDtypeStruct((B, 1, NP), jnp.float32),
        ],
    )(part)


_CH_C = 2000


_NCH_C = EPT // _CH_C


@functools.partial(
    pl.kernel,
    out_type=(
        jax.ShapeDtypeStruct((E,), jnp.float32),
        jax.ShapeDtypeStruct((E,), jnp.int32),
        jax.ShapeDtypeStruct((NTILES * NP,), jnp.float32),
    ),
    scratch_types=[
        pltpu.VMEM((NP_PAD,), jnp.float32),
        pltpu.VMEM((NP_PAD,), jnp.float32),
        pltpu.VMEM((NP_PAD,), jnp.float32),
        pltpu.VMEM((_CH_C,), jnp.int32),
        pltpu.VMEM((_CH_C,), jnp.int32),
        pltpu.VMEM((_CH_C,), jnp.float32),
        pltpu.VMEM((_CH_C,), jnp.int32),
        pltpu.VMEM((_CH_C,), jnp.int32),
        pltpu.VMEM((_CH_C,), jnp.float32),
        pltpu.VMEM((_CH_C,), jnp.float32),
        pltpu.VMEM((_CH_C,), jnp.int32),
        pltpu.VMEM((_CH_C,), jnp.float32),
        pltpu.VMEM((_CH_C,), jnp.int32),
        pltpu.SemaphoreType.DMA,
        pltpu.SemaphoreType.DMA,
        pltpu.SemaphoreType.DMA,
        pltpu.SemaphoreType.DMA,
    ],
    mesh=_MESH,
    compiler_params=pltpu.CompilerParams(needs_layout_passes=False),
)
def _norm_kernel(ei_hbm, ew_hbm, is_hbm, x_hbm, norm_hbm, pk_hbm, agg0_hbm,
                 is_loc, x_loc, agg0_loc,
                 sb0, db0, wb0, sb1, db1, wb1,
                 nb0, pkb0, nb1, pkb1,
                 semi0, semi1, semo0, semo1):
    t = _wid()
    g = t // 8
    base = g * NP
    e0 = t * EPT
    pltpu.sync_copy(is_hbm.at[pl.ds(base, NP)], is_loc.at[pl.ds(0, NP)])
    pltpu.sync_copy(x_hbm.at[pl.ds(base, NP)], x_loc.at[pl.ds(0, NP)])
    _zero_f32(agg0_loc, NP_PAD)

    def start_in(c, sb, db, wb, sem):
        off = e0 + c * _CH_C
        pltpu.make_async_copy(ei_hbm.at[pl.ds(off, _CH_C)], sb, sem).start()
        pltpu.make_async_copy(
            ei_hbm.at[pl.ds(E + off, _CH_C)], db, sem).start()
        pltpu.make_async_copy(ew_hbm.at[pl.ds(off, _CH_C)], wb, sem).start()

    def wait_in(sb, db, wb, sem):
        pltpu.make_async_copy(ei_hbm.at[pl.ds(0, _CH_C)], sb, sem).wait()
        pltpu.make_async_copy(ei_hbm.at[pl.ds(0, _CH_C)], db, sem).wait()
        pltpu.make_async_copy(ew_hbm.at[pl.ds(0, _CH_C)], wb, sem).wait()

    def start_out(c, nb, pkb, sem):
        off = e0 + c * _CH_C
        pltpu.make_async_copy(nb, norm_hbm.at[pl.ds(off, _CH_C)], sem).start()
        pltpu.make_async_copy(pkb, pk_hbm.at[pl.ds(off, _CH_C)], sem).start()

    def wait_out(nb, pkb, sem):
        pltpu.make_async_copy(nb, norm_hbm.at[pl.ds(0, _CH_C)], sem).wait()
        pltpu.make_async_copy(pkb, pk_hbm.at[pl.ds(0, _CH_C)], sem).wait()

    def edge_pass(sb, db, wb, nb, pkb):
        @plsc.parallel_loop(0, _CH_C // 16, unroll=16)
        def inner(i):
            sl = pl.ds(i * 16, 16)
            s = sb[sl] - base
            d = db[sl] - base
            pkb[sl] = s | (d << 16)
            a = plsc.load_gather(is_loc, [s])
            b = plsc.load_gather(is_loc, [d])
            nv = wb[sl] * a * b
            nb[sl] = nv
            xv = plsc.load_gather(x_loc, [s])
            plsc.addupdate_scatter(agg0_loc, [d], nv * xv)

    start_in(0, sb0, db0, wb0, semi0)

    def pair(c2, _):
        c = 2 * c2
        start_in(c + 1, sb1, db1, wb1, semi1)
        wait_in(sb0, db0, wb0, semi0)

        @pl.when(c2 > 0)
        def _():
            wait_out(nb0, pkb0, semo0)

        edge_pass(sb0, db0, wb0, nb0, pkb0)
        start_out(c, nb0, pkb0, semo0)

        @pl.when(c + 2 < _NCH_C)
        def _():
            start_in(c + 2, sb0, db0, wb0, semi0)

        wait_in(sb1, db1, wb1, semi1)

        @pl.when(c2 > 0)
        def _():
            wait_out(nb1, pkb1, semo1)

        edge_pass(sb1, db1, wb1, nb1, pkb1)
        start_out(c + 1, nb1, pkb1, semo1)
        return 0

    lax.fori_loop(0, _NCH_C // 2, pair, 0)
    wait_in(sb0, db0, wb0, semi0)
    wait_out(nb0, pkb0, semo0)
    edge_pass(sb0, db0, wb0, nb0, pkb0)
    start_out(_NCH_C - 1, nb0, pkb0, semo0)
    wait_out(nb0, pkb0, semo0)
    wait_out(nb1, pkb1, semo1)
    pltpu.sync_copy(agg0_loc.at[pl.ds(0, NP)],
                    agg0_hbm.at[pl.ds(t * NP, NP)])


def _layer0_body(part_ref, x_ref, sc_ref, w0_ref, b0_ref, out_ref):
    agg = (jnp.sum(part_ref[...], axis=1, keepdims=True)
           + sc_ref[...] * x_ref[...])
    h = w0_ref[...] * agg + b0_ref[...]
    out_ref[...] = jnp.maximum(h, 0.0)


def _layer0(agg0_part, x3, selfc3, w0col, b0col):
    part = agg0_part.reshape(B, 8, NP)
    return pl.pallas_call(
        _layer0_body,
        grid=(B,),
        in_specs=[
            pl.BlockSpec((1, 8, NP), lambda g: (g, 0, 0)),
            pl.BlockSpec((1, 1, NP), lambda g: (g, 0, 0)),
            pl.BlockSpec((1, 1, NP), lambda g: (g, 0, 0)),
            pl.BlockSpec((F, 1), lambda g: (0, 0)),
            pl.BlockSpec((F, 1), lambda g: (0, 0)),
        ],
        out_specs=pl.BlockSpec((1, F, NP), lambda g: (g, 0, 0)),
        out_shape=jax.ShapeDtypeStruct((B, F, NP), jnp.float32),
    )(part, x3, selfc3, w0col, b0col)


_CH_E = 4000
_NCH_E = EP // _CH_E
_NPAIR = _NCH_E // 2


@functools.partial(
    pl.kernel,
    out_type=jax.ShapeDtypeStruct((B * F * NP,), jnp.float32),
    scratch_types=[
        pltpu.VMEM((NP_PAD,), jnp.float32),
        pltpu.VMEM((NP_PAD,), jnp.float32),
        pltpu.VMEM((NP_PAD,), jnp.float32),
        pltpu.VMEM((NP_PAD,), jnp.float32),
        pltpu.VMEM((_CH_E,), jnp.int32),
        pltpu.VMEM((_CH_E,), jnp.float32),
        pltpu.VMEM((_CH_E,), jnp.int32),
        pltpu.VMEM((_CH_E,), jnp.float32),
        pltpu.SemaphoreType.DMA,
        pltpu.SemaphoreType.DMA,
    ],
    mesh=_MESH,
    compiler_params=pltpu.CompilerParams(needs_layout_passes=False),
)
def _agg_kernel(ht_hbm, pk_hbm, nrm_hbm, out_hbm,
                h0_loc, h1_loc, a0_loc, a1_loc,
                pb0, nb0, pb1, nb1, sem0, sem1):
    t = _wid()
    g = t // 8
    p = t % 8
    r0 = (g * F + 2 * p) * NP
    pltpu.sync_copy(ht_hbm.at[pl.ds(r0, NP)], h0_loc.at[pl.ds(0, NP)])
    pltpu.sync_copy(ht_hbm.at[pl.ds(r0 + NP, NP)], h1_loc.at[pl.ds(0, NP)])
    _zero_f32(a0_loc, NP_PAD)
    _zero_f32(a1_loc, NP_PAD)

    def start(c, pb, nb, sem):
        off = g * EP + c * _CH_E
        pltpu.make_async_copy(pk_hbm.at[pl.ds(off, _CH_E)], pb, sem).start()
        pltpu.make_async_copy(nrm_hbm.at[pl.ds(off, _CH_E)], nb, sem).start()

    def wait(pb, nb, sem):
        pltpu.make_async_copy(pk_hbm.at[pl.ds(0, _CH_E)], pb, sem).wait()
        pltpu.make_async_copy(nrm_hbm.at[pl.ds(0, _CH_E)], nb, sem).wait()

    def edge_pass(pb, nb):
        @plsc.parallel_loop(0, _CH_E // 16, unroll=4)
        def inner(i):
            sl = pl.ds(i * 16, 16)
            pk = pb[sl]
            s = pk & 0xFFFF
            d = pk >> 16
            nv = nb[sl]
            m0 = plsc.load_gather(h0_loc, [s]) * nv
            m1 = plsc.load_gather(h1_loc, [s]) * nv
            plsc.addupdate_scatter(a0_loc, [d], m0)
            plsc.addupdate_scatter(a1_loc, [d], m1)

    start(0, pb0, nb0, sem0)

    def pair(c2, _):
        c = 2 * c2
        start(c + 1, pb1, nb1, sem1)
        wait(pb0, nb0, sem0)
        edge_pass(pb0, nb0)

        @pl.when(c2 + 1 < _NPAIR)
        def _():
            start(c + 2, pb0, nb0, sem0)

        wait(pb1, nb1, sem1)
        edge_pass(pb1, nb1)
        return 0

    lax.fori_loop(0, _NPAIR, pair, 0)
    pltpu.sync_copy(a0_loc.at[pl.ds(0, NP)], out_hbm.at[pl.ds(r0, NP)])
    pltpu.sync_copy(a1_loc.at[pl.ds(0, NP)], out_hbm.at[pl.ds(r0 + NP, NP)])


def _mm_body(agg_ref, ht_ref, sc_ref, wt_ref, b_ref, out_ref, *, relu):
    a2 = agg_ref[...].reshape(F, NP)
    h2 = ht_ref[...].reshape(F, NP)
    sc2 = sc_ref[...].reshape(1, NP)
    tmp = a2 + sc2 * h2
    h = jnp.dot(wt_ref[...], tmp, preferred_element_type=jnp.float32)
    h = h + b_ref[...]
    if relu:
        h = jnp.maximum(h, 0.0)
    out_ref[...] = h[None]


def _matmul(aggT, hT, selfc3, wt, bcol, relu):
    return pl.pallas_call(
        functools.partial(_mm_body, relu=relu),
        grid=(B,),
        in_specs=[
            pl.BlockSpec((1, F, NP), lambda g: (g, 0, 0)),
            pl.BlockSpec((1, F, NP), lambda g: (g, 0, 0)),
            pl.BlockSpec((1, 1, NP), lambda g: (g, 0, 0)),
            pl.BlockSpec((F, F), lambda g: (0, 0)),
            pl.BlockSpec((F, 1), lambda g: (0, 0)),
        ],
        out_specs=pl.BlockSpec((1, F, NP), lambda g: (g, 0, 0)),
        out_shape=jax.ShapeDtypeStruct((B, F, NP), jnp.float32),
    )(aggT, hT, selfc3, wt, bcol)


def _mm_final_body(agg_ref, ht_ref, sc_ref, wt_ref, b_ref, out_ref, gs_ref):
    a2 = agg_ref[...].reshape(F, NP)
    h2 = ht_ref[...].reshape(F, NP)
    sc2 = sc_ref[...].reshape(1, NP)
    tmp = a2 + sc2 * h2
    h = jnp.dot(wt_ref[...], tmp, preferred_element_type=jnp.float32)
    h = h + b_ref[...]
    out_ref[...] = h[None]
    gs_ref[...] = jnp.sum(h, axis=1, keepdims=True)[None]


def _matmul_final(aggT, hT, selfc3, wt, bcol):
    return pl.pallas_call(
        _mm_final_body,
        grid=(B,),
        in_specs=[
            pl.BlockSpec((1, F, NP), lambda g: (g, 0, 0)),
            pl.BlockSpec((1, F, NP), lambda g: (g, 0, 0)),
            pl.BlockSpec((1, 1, NP), lambda g: (g, 0, 0)),
            pl.BlockSpec((F, F), lambda g: (0, 0)),
            pl.BlockSpec((F, 1), lambda g: (0, 0)),
        ],
        out_specs=[
            pl.BlockSpec((1, F, NP), lambda g: (g, 0, 0)),
            pl.BlockSpec((1, F, 1), lambda g: (g, 0, 0)),
        ],
        out_shape=[
            jax.ShapeDtypeStruct((B, F, NP), jnp.float32),
            jax.ShapeDtypeStruct((B, F, 1), jnp.float32),
        ],
    )(aggT, hT, selfc3, wt, bcol)


def _pool_body(ht_ref, gs_ref, watt_ref, out_ref):
    g = pl.program_id(0)
    gmean = gs_ref[...].reshape(B, F) * (1.0 / NP)
    ctx = jnp.tanh(jnp.dot(gmean, watt_ref[...],
                           preferred_element_type=jnp.float32))
    sel = (lax.broadcasted_iota(jnp.int32, (B, 1), 0) == g).astype(jnp.float32)
    cg = jnp.sum(ctx * sel, axis=0, keepdims=True).reshape(F, 1)
    h = ht_ref[...].reshape(F, NP)
    scores = jax.nn.sigmoid(jnp.sum(h * cg, axis=0, keepdims=True))
    gf = jnp.sum(h * scores, axis=1, keepdims=True)
    node = h.T
    gfeat = jnp.broadcast_to(gf.reshape(1, F), (NP, F))
    out_ref[...] = jnp.concatenate([node, gfeat], axis=1)[None]


def _pool(hT3, gsum, watt):
    return pl.pallas_call(
        _pool_body,
        grid=(B,),
        in_specs=[
            pl.BlockSpec((1, F, NP), lambda g: (g, 0, 0)),
            pl.BlockSpec((B, F, 1), lambda g: (0, 0, 0)),
            pl.BlockSpec((F, F), lambda g: (0, 0)),
        ],
        out_specs=pl.BlockSpec((1, NP, 2 * F), lambda g: (g, 0, 0)),
        out_shape=jax.ShapeDtypeStruct((B, NP, 2 * F), jnp.float32),
    )(hT3, gsum, watt)


def kernel(x, edge_index, edge_weight, batch, W0, b0, Wh, bh, W_att):
    assert x.shape == (N, 1) and edge_index.shape == (2, E)
    LH = Wh.shape[0]

    ei_flat = edge_index.reshape(2 * E)
    x_flat = x.reshape(N)
    x3 = x.reshape(B, 1, NP)

    deg_part = _deg_kernel(ei_flat, edge_weight)
    is3, selfc3 = _merge_deg(deg_part)
    is_flat = is3.reshape(N)

    norm, pk, agg0_part = _norm_kernel(ei_flat, edge_weight, is_flat, x_flat)

    w0col = W0.reshape(F, 1)
    b0col = b0.reshape(F, 1)
    hT3 = _layer0(agg0_part, x3, selfc3, w0col, b0col)

    for i in range(LH):
        aggT_flat = _agg_kernel(hT3.reshape(B * F * NP), pk, norm)
        aggT3 = aggT_flat.reshape(B, F, NP)
        wt = Wh[i].T
        bcol = bh[i].reshape(F, 1)
        if i < LH - 1:
            hT3 = _matmul(aggT3, hT3, selfc3, wt, bcol, relu=True)
        else:
            hT3, gsum = _matmul_final(aggT3, hT3, selfc3, wt, bcol)

    state = _pool(hT3, gsum, W_att)
    return state
